# baseline (speedup 1.0000x reference)
import functools

import jax
import jax.numpy as jnp
from jax import lax
from jax.experimental import pallas as pl
from jax.experimental.pallas import tpu as pltpu
from jax.experimental.pallas import tpu_sc as plsc

U = 10000
NI = 10000
D = 128
E = 320000
N2 = U + NI
NC = 2
NS = 16
NW = NC * NS
L = 16
EPW = E // NW
K = 80
NCHK = EPW // K
NPH = 5
NCHP = NCHK // NPH
NPHS = 10
PHB = 32
NCHT = E // K
CAPC = 250
CAPB = 264
NPAD = 4280
UPAD = 10240
TOTCH = E // K
ACTB = 144
WIN = 24
F32 = jnp.float32
I32 = jnp.int32

_MESH = dict(core_axis_name="c", subcore_axis_name="s", num_cores=NC,
             num_subcores=NS)


def _zero_vec(ref, n):
    z = jnp.zeros((L,), ref.dtype)

    def body(i, carry):
        ref[pl.ds(i * L, L)] = z
        return carry

    lax.fori_loop(0, n // L, body, 0)


def _zero_rows(ref, rows):
    z = jnp.zeros((L,), F32)

    def body(i, carry):
        for d in range(D // L):
            ref[i, pl.ds(d * L, L)] = z
        return carry

    lax.fori_loop(0, rows, body, 0)


def _zero_acc_slice(acc, gbuf, base, total):
    _zero_rows(gbuf, K)
    off = 0
    while off < total:
        step = min(K, total - off)
        pltpu.sync_copy(gbuf.at[pl.ds(0, step)],
                        acc.at[pl.ds(base + off, step)])
        off += step


def _scale_rows(gbuf, vals, j):
    def body(g, carry):
        vv = vals[j, pl.ds(g * L, L)]
        for e in range(L):
            vb = jnp.broadcast_to(vv[e], (L,))
            row = g * L + e
            for d in range(D // L):
                gbuf[row, pl.ds(d * L, L)] = gbuf[row, pl.ds(d * L, L)] * vb
        return carry

    lax.fori_loop(0, K // L, body, 0)


def _spread_copy(src_fn, dst_fn, s, total):
    bq = (total // NS) // 8 * 8
    rem = total - NS * bq
    base = s * bq
    pltpu.sync_copy(src_fn(base, bq), dst_fn(base, bq))
    if rem:
        @pl.when(s == NS - 1)
        def _():
            pltpu.sync_copy(src_fn(NS * bq, rem), dst_fn(NS * bq, rem))


def _zero_acc(acc, gbuf, s, total):
    _zero_rows(gbuf, K)
    bq = (total // NS) // 8 * 8
    rem = total - NS * bq
    base = s * bq
    off = 0
    while off < bq:
        step = min(K, bq - off)
        pltpu.sync_copy(gbuf.at[pl.ds(0, step)],
                        acc.at[pl.ds(base + off, step)])
        off += step
    if rem:
        @pl.when(s == NS - 1)
        def _():
            pltpu.sync_copy(gbuf.at[pl.ds(0, rem)],
                            acc.at[pl.ds(NS * bq, rem)])


def _spmm_partial(row3, col3, x, val3=None):
    scale = val3 is not None

    scratch = [
        pltpu.VMEM((NCHP, K), I32),
        pltpu.VMEM((NCHP, K), I32),
        pltpu.VMEM((NCHP, K), F32),
        pltpu.VMEM((K, D), F32),
        pltpu.VMEM((K, D), F32),
        pltpu.VMEM_SHARED((U, D), F32),
        pltpu.SemaphoreType.DMA,
        pltpu.SemaphoreType.DMA,
    ]

    def body(row_h, col_h, val_h, x_h, out_h, rowall, colall, valall, gbufa,
             gbufb, acc, sema, semb):
        c = lax.axis_index("c")
        s = lax.axis_index("s")
        w = s * NC + c
        _zero_acc(acc, gbufa, s, U)
        plsc.subcore_barrier()
        bufs = ((gbufa, sema), (gbufb, semb))

        def proc(j, buf):
            if scale:
                _scale_rows(buf, valall, j)
            pltpu.sync_copy(buf, acc.at[rowall.at[j]], add=True)

        for ph in range(NPH):
            pltpu.sync_copy(row_h.at[w, ph], rowall)
            pltpu.sync_copy(col_h.at[w, ph], colall)
            if scale:
                pltpu.sync_copy(val_h.at[w, ph], valall)
            pltpu.async_copy(x_h.at[colall.at[0]], gbufa, sema)

            def pair(p, carry):
                j0 = 2 * p
                pltpu.make_async_copy(x_h.at[colall.at[j0]], gbufa,
                                      sema).wait()
                pltpu.async_copy(x_h.at[colall.at[j0 + 1]], gbufb, semb)
                proc(j0, gbufa)
                pltpu.make_async_copy(x_h.at[colall.at[j0 + 1]], gbufb,
                                      semb).wait()
                pltpu.async_copy(x_h.at[colall.at[j0 + 2]], gbufa, sema)
                proc(j0 + 1, gbufb)
                return carry

            lax.fori_loop(0, (NCHP - 1) // 2, pair, 0)
            pltpu.make_async_copy(x_h.at[colall.at[NCHP - 1]], gbufa,
                                  sema).wait()
            proc(NCHP - 1, gbufa)
        plsc.subcore_barrier()
        _spread_copy(lambda o, n: acc.at[pl.ds(o, n)],
                     lambda o, n: out_h.at[c, pl.ds(o, n)], s, U)

    args = [row3, col3, val3 if scale else col3, x]
    kern = pl.kernel(
        body,
        out_type=jax.ShapeDtypeStruct((NC, U, D), F32),
        mesh=plsc.VectorSubcoreMesh(**_MESH),
        scratch_types=scratch,
        compiler_params=pltpu.CompilerParams(needs_layout_passes=False),
    )
    return kern(*args)


def _spmm_split(rowp, colp, valp, x, bnd):
    scratch = [
        pltpu.VMEM((PHB, K), I32),
        pltpu.VMEM((PHB, K), I32),
        pltpu.VMEM((PHB, K), F32),
        pltpu.VMEM((K, D), F32),
        pltpu.VMEM((K, D), F32),
        pltpu.VMEM((K,), I32),
        pltpu.VMEM((L,), I32),
        pltpu.VMEM_SHARED((U + L, D), F32),
        pltpu.SemaphoreType.DMA,
        pltpu.SemaphoreType.DMA,
    ]

    def body(row_h, col_h, val_h, x_h, bnd_h, out_h, rowall, colall, valall,
             gbufa, gbufb, rowloc, bndv, acc, sema, semb):
        c = lax.axis_index("c")
        s = lax.axis_index("s")
        pltpu.sync_copy(bnd_h, bndv)
        b1 = bndv[pl.ds(0, L)][0]
        lo = jnp.where(c == 0, 0, b1)
        hi = jnp.where(c == 0, b1, E)
        i0 = lo // K
        ihi = (hi + K - 1) // K
        cpt = (ihi - i0 + NS - 1) // NS
        start = i0 + s * cpt
        _zero_acc(acc, gbufa, s, U + L)
        plsc.subcore_barrier()
        rbase = c * U

        for ph in range(NPHS):
            start_ph = start + ph * NCHP
            start8 = start_ph // 8 * 8
            doff = start_ph - start8

            @pl.when(ph * NCHP < cpt)
            def _():
                pltpu.sync_copy(row_h.at[pl.ds(start8, PHB)], rowall)
                pltpu.sync_copy(col_h.at[pl.ds(start8, PHB)], colall)
                pltpu.sync_copy(val_h.at[pl.ds(start8, PHB)], valall)

                def pred(t):
                    return jnp.logical_and(ph * NCHP + t < cpt,
                                           (start_ph + t) * K < hi)

                def issue(t, buf, sem):
                    @pl.when(pred(t))
                    def _():
                        pltpu.async_copy(x_h.at[colall.at[t + doff]], buf,
                                         sem)

                def proc(t, buf, sem):
                    @pl.when(pred(t))
                    def _():
                        tt = t + doff
                        pltpu.make_async_copy(x_h.at[colall.at[tt]], buf,
                                              sem).wait()
                        _scale_rows(buf, valall, tt)
                        for g in range(K // L):
                            r = rowall[tt, pl.ds(g * L, L)]
                            local = r - rbase
                            inb = jnp.logical_and(local >= 0, local < U)
                            rowloc[pl.ds(g * L, L)] = jnp.where(inb, local, U)
                        pltpu.sync_copy(buf, acc.at[rowloc], add=True)

                issue(0, gbufa, sema)

                def pair(p, carry):
                    t0 = 2 * p
                    issue(t0 + 1, gbufb, semb)
                    proc(t0, gbufa, sema)
                    issue(t0 + 2, gbufa, sema)
                    proc(t0 + 1, gbufb, semb)
                    return carry

                lax.fori_loop(0, (NCHP - 1) // 2, pair, 0)
                proc(NCHP - 1, gbufa, sema)
        plsc.subcore_barrier()
        _spread_copy(lambda o, n: acc.at[pl.ds(o, n)],
                     lambda o, n: out_h.at[pl.ds(c * U + o, n)], s, U)

    kern = pl.kernel(
        body,
        out_type=jax.ShapeDtypeStruct((N2, D), F32),
        mesh=plsc.VectorSubcoreMesh(**_MESH),
        scratch_types=scratch,
        compiler_params=pltpu.CompilerParams(needs_layout_passes=False),
    )
    return kern(rowp, colp, valp, x, bnd)


def _sims_kernel(row3, col3, a_norm):
    scratch = [
        pltpu.VMEM((NCHP, K), I32),
        pltpu.VMEM((NCHP, K), I32),
        pltpu.VMEM((NCHP, K), F32),
        pltpu.VMEM((K, D), F32),
        pltpu.VMEM((K, D), F32),
        pltpu.VMEM((K, D), F32),
        pltpu.VMEM((K, D), F32),
        pltpu.VMEM((WIN, D), F32),
        pltpu.VMEM((WIN, D), F32),
        pltpu.VMEM((ACTB,), I32),
        pltpu.VMEM((L,), I32),
        pltpu.VMEM_SHARED((UPAD,), F32),
        pltpu.SemaphoreType.DMA,
        pltpu.SemaphoreType.DMA,
    ]

    def body(row_h, col_h, an_h, prn_h, diag_h, act_h, cnt_h, rowall, colall,
             prnall, browa, bcola, browb, bcolb, wina, winb, actb, cntb, dacc,
             sem1, sem2):
        c = lax.axis_index("c")
        s = lax.axis_index("s")
        w = s * NC + c
        _zero_vec(prnall.at[0], K)
        dpt = UPAD // NS
        for q in range(dpt // K):
            pltpu.sync_copy(prnall.at[0],
                            dacc.at[pl.ds(s * dpt + q * K, K)])
        plsc.subcore_barrier()
        lanes = lax.iota(I32, L)

        def chunk_meta(j):
            r0 = rowall[j, pl.ds(0, L)][0]
            rlast = rowall[j, pl.ds(K - L, L)][L - 1]
            w0 = jnp.minimum(r0 // 8 * 8, U - WIN)
            return w0, rlast < w0 + WIN

        def issue(j, win, brow, bcol, sem):
            w0, small = chunk_meta(j)

            @pl.when(small)
            def _():
                pltpu.async_copy(an_h.at[pl.ds(w0, WIN)], win, sem)

            @pl.when(jnp.logical_not(small))
            def _():
                pltpu.async_copy(an_h.at[rowall.at[j]], brow, sem)

            pltpu.async_copy(an_h.at[colall.at[j]], bcol, sem)

        def drain(j, win, brow, bcol, sem):
            w0, small = chunk_meta(j)

            @pl.when(small)
            def _():
                pltpu.make_async_copy(an_h.at[pl.ds(w0, WIN)], win,
                                      sem).wait()

            @pl.when(jnp.logical_not(small))
            def _():
                pltpu.make_async_copy(an_h.at[rowall.at[j]], brow,
                                      sem).wait()

            pltpu.make_async_copy(an_h.at[colall.at[j]], bcol, sem).wait()

        def phase(ph, cursor):
            pltpu.sync_copy(row_h.at[w, ph], rowall)
            pltpu.sync_copy(col_h.at[w, ph], colall)
            issue(0, wina, browa, bcola, sem1)

            def compute(j, cur, win, brow, bcol):
                w0, small = chunk_meta(j)

                def mkgroup(rowref, use_win):
                    def group(g, nsurv):
                        rr = rowall[j, pl.ds(g * L, L)] - w0
                        dots = jnp.zeros((L,), F32)
                        for e in range(L):
                            erow = rr[e] if use_win else g * L + e
                            part = jnp.zeros((L,), F32)
                            for d in range(D // L):
                                part = (part
                                        + rowref[erow, pl.ds(d * L, L)]
                                        * bcol[g * L + e, pl.ds(d * L, L)])
                            dot = jnp.sum(part)
                            dots = jnp.where(lanes == e, dot, dots)
                        simv = (dots + 1.0) * 0.5
                        keep = simv >= 0.8
                        pv = jnp.where(keep, simv, 0.0)
                        prnall[j, pl.ds(g * L, L)] = pv
                        return nsurv + jnp.sum(jnp.where(keep, 1, 0))
                    return group

                nsurv = lax.cond(
                    small,
                    lambda: lax.fori_loop(0, K // L, mkgroup(win, True), 0),
                    lambda: lax.fori_loop(0, K // L, mkgroup(brow, False), 0))
                pltpu.sync_copy(prnall.at[j], dacc.at[rowall.at[j]], add=True)
                jg = ph * NCHP + j
                plsc.store_compressed(actb.at[pl.ds(cur, L)],
                                      jnp.broadcast_to(jg, (L,)),
                                      mask=lanes == 0)
                return cur + jnp.where(nsurv > 0, 1, 0)

            def pair(p, cur):
                j0 = 2 * p
                drain(j0, wina, browa, bcola, sem1)
                issue(j0 + 1, winb, browb, bcolb, sem2)
                cur = compute(j0, cur, wina, browa, bcola)
                drain(j0 + 1, winb, browb, bcolb, sem2)
                issue(j0 + 2, wina, browa, bcola, sem1)
                cur = compute(j0 + 1, cur, winb, browb, bcolb)
                return cur

            cur = lax.fori_loop(0, (NCHP - 1) // 2, pair, cursor)
            drain(NCHP - 1, wina, browa, bcola, sem1)
            cur = compute(NCHP - 1, cur, wina, browa, bcola)
            pltpu.sync_copy(prnall, prn_h.at[w, ph])
            return cur

        cursor = lax.fori_loop(0, NPH, phase, 0)
        cntb[pl.ds(0, L)] = jnp.where(lanes == 0, cursor, 0)
        pltpu.sync_copy(actb, act_h.at[w])
        pltpu.sync_copy(cntb, cnt_h.at[w])
        plsc.subcore_barrier()
        pltpu.sync_copy(dacc.at[pl.ds(s * dpt, dpt)],
                        diag_h.at[c, pl.ds(s * dpt, dpt)])

    kern = pl.kernel(
        body,
        out_type=[jax.ShapeDtypeStruct((NW, NPH, NCHP, K), F32),
                  jax.ShapeDtypeStruct((NC, UPAD), F32),
                  jax.ShapeDtypeStruct((NW, ACTB), I32),
                  jax.ShapeDtypeStruct((NW, L), I32)],
        mesh=plsc.VectorSubcoreMesh(**_MESH),
        scratch_types=scratch,
        compiler_params=pltpu.CompilerParams(needs_layout_passes=False),
    )
    return kern(row3, col3, a_norm)


def _social_chain(rowf, colf, valf, act, cnt, x0, diag0, diag1, two):
    scratch = [
        pltpu.VMEM((ACTB,), I32),
        pltpu.VMEM((L,), I32),
        pltpu.VMEM((K,), I32),
        pltpu.VMEM((K,), I32),
        pltpu.VMEM((K,), I32),
        pltpu.VMEM((K,), F32),
        pltpu.VMEM((K,), F32),
        pltpu.VMEM((K,), F32),
        pltpu.VMEM((K, D), F32),
        pltpu.VMEM_SHARED((U, D), F32),
        pltpu.SemaphoreType.DMA,
    ]

    def mkbody(*refs):
        if two:
            (row_h, col_h, val_h, act_h, cnt_h, x0_h, dg0_h, dg1_h,
             u1_h, u2_h, actb, cntb, rowb, colb, colb2, valb,
             d0b, d1b, gbuf, acc, sem) = refs
        else:
            (row_h, col_h, val_h, act_h, cnt_h, x0_h, dg0_h, dg1_h,
             u1_h, actb, cntb, rowb, colb, colb2, valb,
             d0b, d1b, gbuf, acc, sem) = refs
            u2_h = u1_h
        c = lax.axis_index("c")
        s = lax.axis_index("s")
        lanes = lax.iota(I32, L)
        cbase = c * U

        def layer(src_h, dst_h, first):
            _zero_acc(acc, gbuf, s, U)
            plsc.subcore_barrier()

            def per_worker(wi, carry):
                w = s + NS * wi
                pltpu.sync_copy(act_h.at[w], actb)
                pltpu.sync_copy(cnt_h.at[w], cntb)
                nact = jnp.sum(jnp.where(lanes == 0, cntb[pl.ds(0, L)], 0))

                def agroup(g, carry2):
                    av = actb[pl.ds(g * L, L)]
                    for e in range(L):
                        jloc = av[e]

                        @pl.when(g * L + e < nact)
                        def _():
                            gcid = w * NCHK + jloc
                            pltpu.sync_copy(row_h.at[gcid, 0], rowb)
                            pltpu.sync_copy(col_h.at[gcid, 0], colb)
                            pltpu.sync_copy(val_h.at[gcid, 0], valb)
                            pltpu.async_copy(dg0_h.at[rowb], d0b, sem).wait()
                            pltpu.async_copy(dg1_h.at[rowb], d1b, sem).wait()
                            for gg in range(K // L):
                                sl = pl.ds(gg * L, L)
                                valb[sl] = valb[sl] / (d0b[sl] + d1b[sl]
                                                       + 1e-7)
                                if not first:
                                    colb2[sl] = colb[sl] + cbase
                            if first:
                                pltpu.async_copy(src_h.at[colb], gbuf,
                                                 sem).wait()
                            else:
                                pltpu.async_copy(src_h.at[colb2], gbuf,
                                                 sem).wait()

                            def sgroup(gg, c2):
                                vv = valb[pl.ds(gg * L, L)]
                                for ee in range(L):
                                    vb = jnp.broadcast_to(vv[ee], (L,))
                                    r = gg * L + ee
                                    for d in range(D // L):
                                        gbuf[r, pl.ds(d * L, L)] = (
                                            gbuf[r, pl.ds(d * L, L)] * vb)
                                return c2

                            lax.fori_loop(0, K // L, sgroup, 0)
                            pltpu.sync_copy(gbuf, acc.at[rowb], add=True)

                    return carry2

                lax.fori_loop(0, ACTB // L, agroup, 0)
                return carry

            lax.fori_loop(0, 2, per_worker, 0)
            plsc.subcore_barrier()
            _spread_copy(lambda o, n: acc.at[pl.ds(o, n)],
                         lambda o, n: dst_h.at[pl.ds(cbase + o, n)], s, U)
            plsc.subcore_barrier()

        if two:
            layer(x0_h, u1_h, False)
            layer(u1_h, u2_h, False)
        else:
            layer(x0_h, u1_h, True)

    out = jax.ShapeDtypeStruct((NC * U, D), F32)
    kern = pl.kernel(
        mkbody,
        out_type=[out, out] if two else [out],
        mesh=plsc.VectorSubcoreMesh(**_MESH),
        scratch_types=scratch,
        compiler_params=pltpu.CompilerParams(needs_layout_passes=False),
    )
    return kern(rowf, colf, valf, act, cnt, x0, diag0, diag1)


_BU = 1000


def _tc_normalize(u2p):
    def body(pref, oref):
        x = pref[0] + pref[1]
        n = jnp.sqrt(jnp.sum(x * x, axis=1, keepdims=True))
        oref[...] = x / jnp.maximum(n, 1e-8)

    return pl.pallas_call(
        body,
        grid=(U // _BU,),
        in_specs=[pl.BlockSpec((NC, _BU, D), lambda i: (0, i, 0))],
        out_specs=pl.BlockSpec((_BU, D), lambda i: (i, 0)),
        out_shape=jax.ShapeDtypeStruct((U, D), F32),
    )(u2p)


def _tc_combine_social(pp, diag2):
    def body(pref, dref, oref):
        dsum = dref[0] + dref[1] + 1e-7
        oref[...] = (pref[0] + pref[1]) / dsum

    return pl.pallas_call(
        body,
        grid=(U // _BU,),
        in_specs=[pl.BlockSpec((NC, _BU, D), lambda i: (0, i, 0)),
                  pl.BlockSpec((NC, _BU, 1), lambda i: (0, i, 0))],
        out_specs=pl.BlockSpec((_BU, D), lambda i: (i, 0)),
        out_shape=jax.ShapeDtypeStruct((U, D), F32),
    )(pp, diag2.reshape(NC, U, 1))


def _tc_sview_ego(u1, u2, u3, user_emb):
    def body(a, b, c, ue, sv, eg):
        m = (a[...] + b[...] + c[...]) / 3.0
        sv[...] = m
        eg[...] = ue[...] + m

    bs = pl.BlockSpec((_BU, D), lambda i: (i, 0))
    return pl.pallas_call(
        body,
        grid=(U // _BU,),
        in_specs=[bs, bs, bs, bs],
        out_specs=[bs, bs],
        out_shape=[jax.ShapeDtypeStruct((U, D), F32),
                   jax.ShapeDtypeStruct((U, D), F32)],
    )(u1, u2, u3, user_emb)


def _tc_perturb(raw, noise):
    def body(rref, nref, oref):
        nz = nref[...]
        nn = nz / jnp.maximum(
            jnp.sqrt(jnp.sum(nz * nz, axis=1, keepdims=True)), 1e-12)
        r = rref[...]
        oref[...] = r + jnp.sign(r) * nn * 0.1

    bs = pl.BlockSpec((_BU, D), lambda i: (i, 0))
    return pl.pallas_call(
        body,
        grid=(N2 // _BU,),
        in_specs=[bs, bs],
        out_specs=bs,
        out_shape=jax.ShapeDtypeStruct((N2, D), F32),
    )(raw, noise)


def _tc_final_stack(ego0, ego1, ego2, sview, w1t, w2t):
    def body(a, b, c, sv, w1, w2, ua, iv, st):
        e0, e1, e2 = a[...], b[...], c[...]
        st[...] = jnp.stack([e0, e1, e2], axis=1)
        m = (e0 + e1 + e2) / 3.0
        iv[...] = m
        svv = sv[...]
        z = (jnp.dot(m, w1[...], preferred_element_type=F32)
             + jnp.dot(svv, w2[...], preferred_element_type=F32))
        gu = jax.nn.sigmoid(z)
        ua[...] = gu * svv + (1.0 - gu) * m

    bs = pl.BlockSpec((_BU, D), lambda i: (i, 0))
    us = pl.BlockSpec((_BU, D), lambda i: (jnp.minimum(i, U // _BU - 1), 0))
    ws = pl.BlockSpec((D, D), lambda i: (0, 0))
    ss = pl.BlockSpec((_BU, 3, D), lambda i: (i, 0, 0))
    return pl.pallas_call(
        body,
        grid=(N2 // _BU,),
        in_specs=[bs, bs, bs, us, ws, ws],
        out_specs=[bs, bs, ss],
        out_shape=[jax.ShapeDtypeStruct((N2, D), F32),
                   jax.ShapeDtypeStruct((N2, D), F32),
                   jax.ShapeDtypeStruct((N2, 3, D), F32)],
    )(ego0, ego1, ego2, sview, w1t, w2t)


def kernel(user_emb, item_emb, social_row, social_col, social_val,
           adj_row, adj_col, adj_val, W1, W2):
    del social_val
    srow = social_row.astype(I32)
    scol = social_col.astype(I32)
    arow = adj_row.astype(I32)
    acol = adj_col.astype(I32)
    aval = adj_val.astype(F32)

    srow3 = srow.reshape(NW, NPH, NCHP, K)
    scol3 = scol.reshape(NW, NPH, NCHP, K)

    u2p = _spmm_partial(srow3, scol3, user_emb)
    a_norm = _tc_normalize(u2p)

    pruned3, diag_pad, act, cnt = _sims_kernel(srow3, scol3, a_norm)
    diag0 = diag_pad[0]
    diag1 = diag_pad[1]
    srowf = srow.reshape(TOTCH, 1, K)
    scolf = scol.reshape(TOTCH, 1, K)
    valf = pruned3.reshape(TOTCH, 1, K)

    (up1,) = _social_chain(srowf, scolf, valf, act, cnt, user_emb, diag0,
                           diag1, two=False)
    up2, up3 = _social_chain(srowf, scolf, valf, act, cnt, up1, diag0, diag1,
                             two=True)
    user_sview, ego_user = _tc_sview_ego(up1[:U], up2[:U], up3[:U], user_emb)
    ego0 = jnp.concatenate([ego_user, item_emb], axis=0)

    pad_rows = NPAD - NCHT
    arowp = jnp.concatenate(
        [arow.reshape(NCHT, K), jnp.zeros((pad_rows, K), I32)], axis=0)
    acolp = jnp.concatenate(
        [acol.reshape(NCHT, K), jnp.zeros((pad_rows, K), I32)], axis=0)
    avalp = jnp.concatenate(
        [aval.reshape(NCHT, K), jnp.zeros((pad_rows, K), F32)], axis=0)
    b1 = jnp.searchsorted(arow, U).astype(I32)
    bnd = jnp.broadcast_to(b1, (L,)).astype(I32)

    nkey = jax.random.key(42)
    egos = [ego0]
    ego = ego0
    for k in range(2):
        raw = _spmm_split(arowp, acolp, avalp, ego, bnd)
        noise = jax.random.uniform(jax.random.fold_in(nkey, k), (N2, D),
                                   dtype=F32)
        ego = _tc_perturb(raw, noise)
        egos.append(ego)

    ua_full, iv_full, stack = _tc_final_stack(
        egos[0], egos[1], egos[2], user_sview, W1.T, W2.T)
    return (ua_full[:U], iv_full[U:], stack)


# speedup vs baseline: 1.0399x; 1.0399x over previous
import functools

import jax
import jax.numpy as jnp
from jax import lax
from jax.experimental import pallas as pl
from jax.experimental.pallas import tpu as pltpu
from jax.experimental.pallas import tpu_sc as plsc

U = 10000
NI = 10000
D = 128
E = 320000
N2 = U + NI
NC = 2
NS = 16
NW = NC * NS
L = 16
EPW = E // NW
K = 80
NCHK = EPW // K
NPH = 5
NCHP = NCHK // NPH
NPHS = 10
PHB = 32
NCHT = E // K
CAPC = 250
CAPB = 264
NPAD = 4280
UPAD = 10240
TOTCH = E // K
ACTB = 144
WIN = 24
F32 = jnp.float32
I32 = jnp.int32

_MESH = dict(core_axis_name="c", subcore_axis_name="s", num_cores=NC,
             num_subcores=NS)


def _zero_vec(ref, n):
    z = jnp.zeros((L,), ref.dtype)

    def body(i, carry):
        ref[pl.ds(i * L, L)] = z
        return carry

    lax.fori_loop(0, n // L, body, 0)


def _zero_rows(ref, rows):
    z = jnp.zeros((L,), F32)

    def body(i, carry):
        for d in range(D // L):
            ref[i, pl.ds(d * L, L)] = z
        return carry

    lax.fori_loop(0, rows, body, 0)


def _zero_acc_slice(acc, gbuf, base, total):
    _zero_rows(gbuf, K)
    off = 0
    while off < total:
        step = min(K, total - off)
        pltpu.sync_copy(gbuf.at[pl.ds(0, step)],
                        acc.at[pl.ds(base + off, step)])
        off += step


def _scale_rows(gbuf, vals, j):
    def body(g, carry):
        vv = vals[j, pl.ds(g * L, L)]
        for e in range(L):
            vb = jnp.broadcast_to(vv[e], (L,))
            row = g * L + e
            for d in range(D // L):
                gbuf[row, pl.ds(d * L, L)] = gbuf[row, pl.ds(d * L, L)] * vb
        return carry

    lax.fori_loop(0, K // L, body, 0)


def _spread_copy(src_fn, dst_fn, s, total):
    bq = (total // NS) // 8 * 8
    rem = total - NS * bq
    base = s * bq
    pltpu.sync_copy(src_fn(base, bq), dst_fn(base, bq))
    if rem:
        @pl.when(s == NS - 1)
        def _():
            pltpu.sync_copy(src_fn(NS * bq, rem), dst_fn(NS * bq, rem))


def _zero_acc(acc, gbuf, s, total):
    _zero_rows(gbuf, K)
    bq = (total // NS) // 8 * 8
    rem = total - NS * bq
    base = s * bq
    off = 0
    while off < bq:
        step = min(K, bq - off)
        pltpu.sync_copy(gbuf.at[pl.ds(0, step)],
                        acc.at[pl.ds(base + off, step)])
        off += step
    if rem:
        @pl.when(s == NS - 1)
        def _():
            pltpu.sync_copy(gbuf.at[pl.ds(0, rem)],
                            acc.at[pl.ds(NS * bq, rem)])


def _spmm_partial(row3, col3, x, val3=None):
    scale = val3 is not None

    scratch = [
        pltpu.VMEM((NCHP, K), I32),
        pltpu.VMEM((NCHP, K), I32),
        pltpu.VMEM((NCHP, K), F32),
        pltpu.VMEM((K, D), F32),
        pltpu.VMEM((K, D), F32),
        pltpu.VMEM_SHARED((U, D), F32),
        pltpu.SemaphoreType.DMA,
        pltpu.SemaphoreType.DMA,
    ]

    def body(row_h, col_h, val_h, x_h, out_h, rowall, colall, valall, gbufa,
             gbufb, acc, sema, semb):
        c = lax.axis_index("c")
        s = lax.axis_index("s")
        w = s * NC + c
        _zero_acc(acc, gbufa, s, U)
        plsc.subcore_barrier()
        bufs = ((gbufa, sema), (gbufb, semb))

        def proc(j, buf):
            if scale:
                _scale_rows(buf, valall, j)
            pltpu.sync_copy(buf, acc.at[rowall.at[j]], add=True)

        for ph in range(NPH):
            pltpu.sync_copy(row_h.at[w, ph], rowall)
            pltpu.sync_copy(col_h.at[w, ph], colall)
            if scale:
                pltpu.sync_copy(val_h.at[w, ph], valall)
            pltpu.async_copy(x_h.at[colall.at[0]], gbufa, sema)

            def pair(p, carry):
                j0 = 2 * p
                pltpu.make_async_copy(x_h.at[colall.at[j0]], gbufa,
                                      sema).wait()
                pltpu.async_copy(x_h.at[colall.at[j0 + 1]], gbufb, semb)
                proc(j0, gbufa)
                pltpu.make_async_copy(x_h.at[colall.at[j0 + 1]], gbufb,
                                      semb).wait()
                pltpu.async_copy(x_h.at[colall.at[j0 + 2]], gbufa, sema)
                proc(j0 + 1, gbufb)
                return carry

            lax.fori_loop(0, (NCHP - 1) // 2, pair, 0)
            pltpu.make_async_copy(x_h.at[colall.at[NCHP - 1]], gbufa,
                                  sema).wait()
            proc(NCHP - 1, gbufa)
        plsc.subcore_barrier()
        _spread_copy(lambda o, n: acc.at[pl.ds(o, n)],
                     lambda o, n: out_h.at[c, pl.ds(o, n)], s, U)

    args = [row3, col3, val3 if scale else col3, x]
    kern = pl.kernel(
        body,
        out_type=jax.ShapeDtypeStruct((NC, U, D), F32),
        mesh=plsc.VectorSubcoreMesh(**_MESH),
        scratch_types=scratch,
        compiler_params=pltpu.CompilerParams(needs_layout_passes=False),
    )
    return kern(*args)


def _spmm_split(rowp, colp, valp, x, bnd):
    scratch = [
        pltpu.VMEM((PHB, K), I32),
        pltpu.VMEM((PHB, K), I32),
        pltpu.VMEM((PHB, K), F32),
        pltpu.VMEM((K, D), F32),
        pltpu.VMEM((K, D), F32),
        pltpu.VMEM((K,), I32),
        pltpu.VMEM((L,), I32),
        pltpu.VMEM_SHARED((U + L, D), F32),
        pltpu.SemaphoreType.DMA,
        pltpu.SemaphoreType.DMA,
    ]

    def body(row_h, col_h, val_h, x_h, bnd_h, out_h, rowall, colall, valall,
             gbufa, gbufb, rowloc, bndv, acc, sema, semb):
        c = lax.axis_index("c")
        s = lax.axis_index("s")
        pltpu.sync_copy(bnd_h, bndv)
        b1 = bndv[pl.ds(0, L)][0]
        lo = jnp.where(c == 0, 0, b1)
        hi = jnp.where(c == 0, b1, E)
        i0 = lo // K
        ihi = (hi + K - 1) // K
        cpt = (ihi - i0 + NS - 1) // NS
        start = i0 + s * cpt
        _zero_acc(acc, gbufa, s, U + L)
        plsc.subcore_barrier()
        rbase = c * U

        for ph in range(NPHS):
            start_ph = start + ph * NCHP
            start8 = start_ph // 8 * 8
            doff = start_ph - start8

            @pl.when(ph * NCHP < cpt)
            def _():
                pltpu.sync_copy(row_h.at[pl.ds(start8, PHB)], rowall)
                pltpu.sync_copy(col_h.at[pl.ds(start8, PHB)], colall)
                pltpu.sync_copy(val_h.at[pl.ds(start8, PHB)], valall)

                def pred(t):
                    return jnp.logical_and(ph * NCHP + t < cpt,
                                           (start_ph + t) * K < hi)

                def issue(t, buf, sem):
                    @pl.when(pred(t))
                    def _():
                        pltpu.async_copy(x_h.at[colall.at[t + doff]], buf,
                                         sem)

                def proc(t, buf, sem):
                    @pl.when(pred(t))
                    def _():
                        tt = t + doff
                        pltpu.make_async_copy(x_h.at[colall.at[tt]], buf,
                                              sem).wait()
                        _scale_rows(buf, valall, tt)
                        for g in range(K // L):
                            r = rowall[tt, pl.ds(g * L, L)]
                            local = r - rbase
                            inb = jnp.logical_and(local >= 0, local < U)
                            rowloc[pl.ds(g * L, L)] = jnp.where(inb, local, U)
                        pltpu.sync_copy(buf, acc.at[rowloc], add=True)

                issue(0, gbufa, sema)

                def pair(p, carry):
                    t0 = 2 * p
                    issue(t0 + 1, gbufb, semb)
                    proc(t0, gbufa, sema)
                    issue(t0 + 2, gbufa, sema)
                    proc(t0 + 1, gbufb, semb)
                    return carry

                lax.fori_loop(0, (NCHP - 1) // 2, pair, 0)
                proc(NCHP - 1, gbufa, sema)
        plsc.subcore_barrier()
        _spread_copy(lambda o, n: acc.at[pl.ds(o, n)],
                     lambda o, n: out_h.at[pl.ds(c * U + o, n)], s, U)

    kern = pl.kernel(
        body,
        out_type=jax.ShapeDtypeStruct((N2, D), F32),
        mesh=plsc.VectorSubcoreMesh(**_MESH),
        scratch_types=scratch,
        compiler_params=pltpu.CompilerParams(needs_layout_passes=False),
    )
    return kern(rowp, colp, valp, x, bnd)


def _sims_kernel(row3, col3, a_norm):
    scratch = [
        pltpu.VMEM((NCHP, K), I32),
        pltpu.VMEM((NCHP, K), I32),
        pltpu.VMEM((NCHP, K), F32),
        pltpu.VMEM((K, D), F32),
        pltpu.VMEM((K, D), F32),
        pltpu.VMEM((K, D), F32),
        pltpu.VMEM((K, D), F32),
        pltpu.VMEM((WIN, D), F32),
        pltpu.VMEM((WIN, D), F32),
        pltpu.VMEM((ACTB,), I32),
        pltpu.VMEM((L,), I32),
        pltpu.VMEM_SHARED((UPAD,), F32),
        pltpu.SemaphoreType.DMA,
        pltpu.SemaphoreType.DMA,
    ]

    def body(row_h, col_h, an_h, prn_h, diag_h, act_h, cnt_h, rowall, colall,
             prnall, browa, bcola, browb, bcolb, wina, winb, actb, cntb, dacc,
             sem1, sem2):
        c = lax.axis_index("c")
        s = lax.axis_index("s")
        w = s * NC + c
        _zero_vec(prnall.at[0], K)
        dpt = UPAD // NS
        for q in range(dpt // K):
            pltpu.sync_copy(prnall.at[0],
                            dacc.at[pl.ds(s * dpt + q * K, K)])
        plsc.subcore_barrier()
        lanes = lax.iota(I32, L)

        def chunk_meta(j):
            r0 = rowall[j, pl.ds(0, L)][0]
            rlast = rowall[j, pl.ds(K - L, L)][L - 1]
            w0 = jnp.minimum(r0 // 8 * 8, U - WIN)
            return w0, rlast < w0 + WIN

        def issue(j, win, brow, bcol, sem):
            w0, small = chunk_meta(j)

            @pl.when(small)
            def _():
                pltpu.async_copy(an_h.at[pl.ds(w0, WIN)], win, sem)

            @pl.when(jnp.logical_not(small))
            def _():
                pltpu.async_copy(an_h.at[rowall.at[j]], brow, sem)

            pltpu.async_copy(an_h.at[colall.at[j]], bcol, sem)

        def drain(j, win, brow, bcol, sem):
            w0, small = chunk_meta(j)

            @pl.when(small)
            def _():
                pltpu.make_async_copy(an_h.at[pl.ds(w0, WIN)], win,
                                      sem).wait()

            @pl.when(jnp.logical_not(small))
            def _():
                pltpu.make_async_copy(an_h.at[rowall.at[j]], brow,
                                      sem).wait()

            pltpu.make_async_copy(an_h.at[colall.at[j]], bcol, sem).wait()

        def phase(ph, cursor):
            pltpu.sync_copy(row_h.at[w, ph], rowall)
            pltpu.sync_copy(col_h.at[w, ph], colall)
            issue(0, wina, browa, bcola, sem1)

            def compute(j, cur, win, brow, bcol):
                w0, small = chunk_meta(j)

                def mkgroup(rowref, use_win):
                    def group(g, nsurv):
                        rr = rowall[j, pl.ds(g * L, L)] - w0
                        dots = jnp.zeros((L,), F32)
                        for e in range(L):
                            erow = rr[e] if use_win else g * L + e
                            part = jnp.zeros((L,), F32)
                            for d in range(D // L):
                                part = (part
                                        + rowref[erow, pl.ds(d * L, L)]
                                        * bcol[g * L + e, pl.ds(d * L, L)])
                            dot = jnp.sum(part)
                            dots = jnp.where(lanes == e, dot, dots)
                        simv = (dots + 1.0) * 0.5
                        keep = simv >= 0.8
                        pv = jnp.where(keep, simv, 0.0)
                        prnall[j, pl.ds(g * L, L)] = pv
                        return nsurv + jnp.sum(jnp.where(keep, 1, 0))
                    return group

                nsurv = lax.cond(
                    small,
                    lambda: lax.fori_loop(0, K // L, mkgroup(win, True), 0),
                    lambda: lax.fori_loop(0, K // L, mkgroup(brow, False), 0))
                pltpu.sync_copy(prnall.at[j], dacc.at[rowall.at[j]], add=True)
                jg = ph * NCHP + j
                plsc.store_compressed(actb.at[pl.ds(cur, L)],
                                      jnp.broadcast_to(jg, (L,)),
                                      mask=lanes == 0)
                return cur + jnp.where(nsurv > 0, 1, 0)

            def pair(p, cur):
                j0 = 2 * p
                drain(j0, wina, browa, bcola, sem1)
                issue(j0 + 1, winb, browb, bcolb, sem2)
                cur = compute(j0, cur, wina, browa, bcola)
                drain(j0 + 1, winb, browb, bcolb, sem2)
                issue(j0 + 2, wina, browa, bcola, sem1)
                cur = compute(j0 + 1, cur, winb, browb, bcolb)
                return cur

            cur = lax.fori_loop(0, (NCHP - 1) // 2, pair, cursor)
            drain(NCHP - 1, wina, browa, bcola, sem1)
            cur = compute(NCHP - 1, cur, wina, browa, bcola)
            pltpu.sync_copy(prnall, prn_h.at[w, ph])
            return cur

        cursor = lax.fori_loop(0, NPH, phase, 0)
        cntb[pl.ds(0, L)] = jnp.where(lanes == 0, cursor, 0)
        pltpu.sync_copy(actb, act_h.at[w])
        pltpu.sync_copy(cntb, cnt_h.at[w])
        plsc.subcore_barrier()
        pltpu.sync_copy(dacc.at[pl.ds(s * dpt, dpt)],
                        diag_h.at[c, pl.ds(s * dpt, dpt)])

    kern = pl.kernel(
        body,
        out_type=[jax.ShapeDtypeStruct((NW, NPH, NCHP, K), F32),
                  jax.ShapeDtypeStruct((NC, UPAD), F32),
                  jax.ShapeDtypeStruct((NW, ACTB), I32),
                  jax.ShapeDtypeStruct((NW, L), I32)],
        mesh=plsc.VectorSubcoreMesh(**_MESH),
        scratch_types=scratch,
        compiler_params=pltpu.CompilerParams(needs_layout_passes=False),
    )
    return kern(row3, col3, a_norm)


def _spmm_compact(rowf, colf, valf, act, cnt, x0, x1, diag0, diag1):
    scratch = [
        pltpu.VMEM((ACTB,), I32),
        pltpu.VMEM((L,), I32),
        pltpu.VMEM((K,), I32),
        pltpu.VMEM((K,), I32),
        pltpu.VMEM((K,), F32),
        pltpu.VMEM((K,), F32),
        pltpu.VMEM((K,), F32),
        pltpu.VMEM((K, D), F32),
        pltpu.VMEM_SHARED((U, D), F32),
        pltpu.SemaphoreType.DMA,
    ]

    def body(row_h, col_h, val_h, act_h, cnt_h, x0_h, x1_h, dg0_h, dg1_h,
             out_h, actb, cntb, rowb, colb, valb, d0b, d1b, gbuf, acc, sem):
        c = lax.axis_index("c")
        s = lax.axis_index("s")
        w = s * NC + c
        lanes = lax.iota(I32, L)
        pltpu.sync_copy(act_h.at[w], actb)
        pltpu.sync_copy(cnt_h.at[w], cntb)
        nact = jnp.sum(jnp.where(lanes == 0, cntb[pl.ds(0, L)], 0))
        _zero_acc(acc, gbuf, s, U)
        plsc.subcore_barrier()

        def agroup(g, carry):
            av = actb[pl.ds(g * L, L)]
            for e in range(L):
                jloc = av[e]

                @pl.when(g * L + e < nact)
                def _():
                    gcid = w * NCHK + jloc
                    pltpu.sync_copy(row_h.at[gcid, 0], rowb)
                    pltpu.sync_copy(col_h.at[gcid, 0], colb)
                    pltpu.sync_copy(val_h.at[gcid, 0], valb)
                    pltpu.async_copy(dg0_h.at[rowb], d0b, sem).wait()
                    pltpu.async_copy(dg1_h.at[rowb], d1b, sem).wait()
                    for gg in range(K // L):
                        sl = pl.ds(gg * L, L)
                        valb[sl] = valb[sl] / (d0b[sl] + d1b[sl] + 1e-7)
                    pltpu.async_copy(x0_h.at[colb], gbuf, sem).wait()
                    pltpu.async_copy(x1_h.at[colb], gbuf, sem,
                                     add=True).wait()

                    def sgroup(gg, c2):
                        vv = valb[pl.ds(gg * L, L)]
                        for ee in range(L):
                            vb = jnp.broadcast_to(vv[ee], (L,))
                            r = gg * L + ee
                            for d in range(D // L):
                                gbuf[r, pl.ds(d * L, L)] = (
                                    gbuf[r, pl.ds(d * L, L)] * vb)
                        return c2

                    lax.fori_loop(0, K // L, sgroup, 0)
                    pltpu.sync_copy(gbuf, acc.at[rowb], add=True)

            return carry

        lax.fori_loop(0, ACTB // L, agroup, 0)
        plsc.subcore_barrier()
        _spread_copy(lambda o, n: acc.at[pl.ds(o, n)],
                     lambda o, n: out_h.at[c, pl.ds(o, n)], s, U)

    kern = pl.kernel(
        body,
        out_type=jax.ShapeDtypeStruct((NC, U, D), F32),
        mesh=plsc.VectorSubcoreMesh(**_MESH),
        scratch_types=scratch,
        compiler_params=pltpu.CompilerParams(needs_layout_passes=False),
    )
    return kern(rowf, colf, valf, act, cnt, x0, x1, diag0, diag1)


_BU = 1000


def _tc_normalize(u2p):
    def body(pref, oref):
        x = pref[0] + pref[1]
        n = jnp.sqrt(jnp.sum(x * x, axis=1, keepdims=True))
        oref[...] = x / jnp.maximum(n, 1e-8)

    return pl.pallas_call(
        body,
        grid=(U // _BU,),
        in_specs=[pl.BlockSpec((NC, _BU, D), lambda i: (0, i, 0))],
        out_specs=pl.BlockSpec((_BU, D), lambda i: (i, 0)),
        out_shape=jax.ShapeDtypeStruct((U, D), F32),
    )(u2p)


def _tc_combine_social(pp, diag2):
    def body(pref, dref, oref):
        dsum = dref[0] + dref[1] + 1e-7
        oref[...] = (pref[0] + pref[1]) / dsum

    return pl.pallas_call(
        body,
        grid=(U // _BU,),
        in_specs=[pl.BlockSpec((NC, _BU, D), lambda i: (0, i, 0)),
                  pl.BlockSpec((NC, _BU, 1), lambda i: (0, i, 0))],
        out_specs=pl.BlockSpec((_BU, D), lambda i: (i, 0)),
        out_shape=jax.ShapeDtypeStruct((U, D), F32),
    )(pp, diag2.reshape(NC, U, 1))


def _tc_sview_ego(p1, p2, p3, user_emb):
    def body(a, b, c, ue, sv, eg):
        m = (a[0] + a[1] + b[0] + b[1] + c[0] + c[1]) / 3.0
        sv[...] = m
        eg[...] = ue[...] + m

    ps = pl.BlockSpec((NC, _BU, D), lambda i: (0, i, 0))
    bs = pl.BlockSpec((_BU, D), lambda i: (i, 0))
    return pl.pallas_call(
        body,
        grid=(U // _BU,),
        in_specs=[ps, ps, ps, bs],
        out_specs=[bs, bs],
        out_shape=[jax.ShapeDtypeStruct((U, D), F32),
                   jax.ShapeDtypeStruct((U, D), F32)],
    )(p1, p2, p3, user_emb)


def _tc_perturb(raw, noise):
    def body(rref, nref, oref):
        nz = nref[...]
        nn = nz / jnp.maximum(
            jnp.sqrt(jnp.sum(nz * nz, axis=1, keepdims=True)), 1e-12)
        r = rref[...]
        oref[...] = r + jnp.sign(r) * nn * 0.1

    bs = pl.BlockSpec((_BU, D), lambda i: (i, 0))
    return pl.pallas_call(
        body,
        grid=(N2 // _BU,),
        in_specs=[bs, bs],
        out_specs=bs,
        out_shape=jax.ShapeDtypeStruct((N2, D), F32),
    )(raw, noise)


def _tc_final_stack(ego0, ego1, ego2, sview, w1t, w2t):
    def body(a, b, c, sv, w1, w2, ua, iv, st):
        e0, e1, e2 = a[...], b[...], c[...]
        st[...] = jnp.stack([e0, e1, e2], axis=1)
        m = (e0 + e1 + e2) / 3.0
        iv[...] = m
        svv = sv[...]
        z = (jnp.dot(m, w1[...], preferred_element_type=F32)
             + jnp.dot(svv, w2[...], preferred_element_type=F32))
        gu = jax.nn.sigmoid(z)
        ua[...] = gu * svv + (1.0 - gu) * m

    bs = pl.BlockSpec((_BU, D), lambda i: (i, 0))
    us = pl.BlockSpec((_BU, D), lambda i: (jnp.minimum(i, U // _BU - 1), 0))
    ws = pl.BlockSpec((D, D), lambda i: (0, 0))
    ss = pl.BlockSpec((_BU, 3, D), lambda i: (i, 0, 0))
    return pl.pallas_call(
        body,
        grid=(N2 // _BU,),
        in_specs=[bs, bs, bs, us, ws, ws],
        out_specs=[bs, bs, ss],
        out_shape=[jax.ShapeDtypeStruct((N2, D), F32),
                   jax.ShapeDtypeStruct((N2, D), F32),
                   jax.ShapeDtypeStruct((N2, 3, D), F32)],
    )(ego0, ego1, ego2, sview, w1t, w2t)


def kernel(user_emb, item_emb, social_row, social_col, social_val,
           adj_row, adj_col, adj_val, W1, W2):
    del social_val
    srow = social_row.astype(I32)
    scol = social_col.astype(I32)
    arow = adj_row.astype(I32)
    acol = adj_col.astype(I32)
    aval = adj_val.astype(F32)

    srow3 = srow.reshape(NW, NPH, NCHP, K)
    scol3 = scol.reshape(NW, NPH, NCHP, K)

    u2p = _spmm_partial(srow3, scol3, user_emb)
    a_norm = _tc_normalize(u2p)

    pruned3, diag_pad, act, cnt = _sims_kernel(srow3, scol3, a_norm)
    diag0 = diag_pad[0]
    diag1 = diag_pad[1]
    srowf = srow.reshape(TOTCH, 1, K)
    scolf = scol.reshape(TOTCH, 1, K)
    valf = pruned3.reshape(TOTCH, 1, K)

    x0, x1 = user_emb, jnp.zeros((U, D), F32)
    pps = []
    for _ in range(3):
        pp = _spmm_compact(srowf, scolf, valf, act, cnt, x0, x1, diag0, diag1)
        pps.append(pp)
        x0, x1 = pp[0], pp[1]
    user_sview, ego_user = _tc_sview_ego(pps[0], pps[1], pps[2], user_emb)
    ego0 = jnp.concatenate([ego_user, item_emb], axis=0)

    pad_rows = NPAD - NCHT
    arowp = jnp.concatenate(
        [arow.reshape(NCHT, K), jnp.zeros((pad_rows, K), I32)], axis=0)
    acolp = jnp.concatenate(
        [acol.reshape(NCHT, K), jnp.zeros((pad_rows, K), I32)], axis=0)
    avalp = jnp.concatenate(
        [aval.reshape(NCHT, K), jnp.zeros((pad_rows, K), F32)], axis=0)
    b1 = jnp.searchsorted(arow, U).astype(I32)
    bnd = jnp.broadcast_to(b1, (L,)).astype(I32)

    nkey = jax.random.key(42)
    egos = [ego0]
    ego = ego0
    for k in range(2):
        raw = _spmm_split(arowp, acolp, avalp, ego, bnd)
        noise = jax.random.uniform(jax.random.fold_in(nkey, k), (N2, D),
                                   dtype=F32)
        ego = _tc_perturb(raw, noise)
        egos.append(ego)

    ua_full, iv_full, stack = _tc_final_stack(
        egos[0], egos[1], egos[2], user_sview, W1.T, W2.T)
    return (ua_full[:U], iv_full[U:], stack)


# speedup vs baseline: 1.0422x; 1.0021x over previous
import functools

import jax
import jax.numpy as jnp
from jax import lax
from jax.experimental import pallas as pl
from jax.experimental.pallas import tpu as pltpu
from jax.experimental.pallas import tpu_sc as plsc

U = 10000
NI = 10000
D = 128
E = 320000
N2 = U + NI
NC = 2
NS = 16
NW = NC * NS
L = 16
EPW = E // NW
K = 80
NCHK = EPW // K
NPH = 5
NCHP = NCHK // NPH
NPHS = 10
PHB = 32
NCHT = E // K
CAPC = 250
CAPB = 264
NPAD = 4280
UPAD = 10240
TOTCH = E // K
ACTB = 144
WIN = 24
F32 = jnp.float32
I32 = jnp.int32

_MESH = dict(core_axis_name="c", subcore_axis_name="s", num_cores=NC,
             num_subcores=NS)


def _zero_vec(ref, n):
    z = jnp.zeros((L,), ref.dtype)

    def body(i, carry):
        ref[pl.ds(i * L, L)] = z
        return carry

    lax.fori_loop(0, n // L, body, 0)


def _zero_rows(ref, rows):
    z = jnp.zeros((L,), F32)

    def body(i, carry):
        for d in range(D // L):
            ref[i, pl.ds(d * L, L)] = z
        return carry

    lax.fori_loop(0, rows, body, 0)


def _zero_acc_slice(acc, gbuf, base, total):
    _zero_rows(gbuf, K)
    off = 0
    while off < total:
        step = min(K, total - off)
        pltpu.sync_copy(gbuf.at[pl.ds(0, step)],
                        acc.at[pl.ds(base + off, step)])
        off += step


def _scale_rows(gbuf, vals, j):
    def body(g, carry):
        vv = vals[j, pl.ds(g * L, L)]
        for e in range(L):
            vb = jnp.broadcast_to(vv[e], (L,))
            row = g * L + e
            for d in range(D // L):
                gbuf[row, pl.ds(d * L, L)] = gbuf[row, pl.ds(d * L, L)] * vb
        return carry

    lax.fori_loop(0, K // L, body, 0)


def _spread_copy(src_fn, dst_fn, s, total):
    bq = (total // NS) // 8 * 8
    rem = total - NS * bq
    base = s * bq
    pltpu.sync_copy(src_fn(base, bq), dst_fn(base, bq))
    if rem:
        @pl.when(s == NS - 1)
        def _():
            pltpu.sync_copy(src_fn(NS * bq, rem), dst_fn(NS * bq, rem))


def _zero_acc(acc, gbuf, s, total):
    _zero_rows(gbuf, K)
    bq = (total // NS) // 8 * 8
    rem = total - NS * bq
    base = s * bq
    off = 0
    while off < bq:
        step = min(K, bq - off)
        pltpu.sync_copy(gbuf.at[pl.ds(0, step)],
                        acc.at[pl.ds(base + off, step)])
        off += step
    if rem:
        @pl.when(s == NS - 1)
        def _():
            pltpu.sync_copy(gbuf.at[pl.ds(0, rem)],
                            acc.at[pl.ds(NS * bq, rem)])


def _spmm_partial(row3, col3, x, val3=None):
    scale = val3 is not None

    scratch = [
        pltpu.VMEM((NCHP, K), I32),
        pltpu.VMEM((NCHP, K), I32),
        pltpu.VMEM((NCHP, K), F32),
        pltpu.VMEM((K, D), F32),
        pltpu.VMEM((K, D), F32),
        pltpu.VMEM_SHARED((U, D), F32),
        pltpu.SemaphoreType.DMA,
        pltpu.SemaphoreType.DMA,
    ]

    def body(row_h, col_h, val_h, x_h, out_h, rowall, colall, valall, gbufa,
             gbufb, acc, sema, semb):
        c = lax.axis_index("c")
        s = lax.axis_index("s")
        w = s * NC + c
        _zero_acc(acc, gbufa, s, U)
        plsc.subcore_barrier()
        bufs = ((gbufa, sema), (gbufb, semb))

        def proc(j, buf):
            if scale:
                _scale_rows(buf, valall, j)
            pltpu.sync_copy(buf, acc.at[rowall.at[j]], add=True)

        for ph in range(NPH):
            pltpu.sync_copy(row_h.at[w, ph], rowall)
            pltpu.sync_copy(col_h.at[w, ph], colall)
            if scale:
                pltpu.sync_copy(val_h.at[w, ph], valall)
            pltpu.async_copy(x_h.at[colall.at[0]], gbufa, sema)

            def pair(p, carry):
                j0 = 2 * p
                pltpu.make_async_copy(x_h.at[colall.at[j0]], gbufa,
                                      sema).wait()
                pltpu.async_copy(x_h.at[colall.at[j0 + 1]], gbufb, semb)
                proc(j0, gbufa)
                pltpu.make_async_copy(x_h.at[colall.at[j0 + 1]], gbufb,
                                      semb).wait()
                pltpu.async_copy(x_h.at[colall.at[j0 + 2]], gbufa, sema)
                proc(j0 + 1, gbufb)
                return carry

            lax.fori_loop(0, (NCHP - 1) // 2, pair, 0)
            pltpu.make_async_copy(x_h.at[colall.at[NCHP - 1]], gbufa,
                                  sema).wait()
            proc(NCHP - 1, gbufa)
        plsc.subcore_barrier()
        _spread_copy(lambda o, n: acc.at[pl.ds(o, n)],
                     lambda o, n: out_h.at[c, pl.ds(o, n)], s, U)

    args = [row3, col3, val3 if scale else col3, x]
    kern = pl.kernel(
        body,
        out_type=jax.ShapeDtypeStruct((NC, U, D), F32),
        mesh=plsc.VectorSubcoreMesh(**_MESH),
        scratch_types=scratch,
        compiler_params=pltpu.CompilerParams(needs_layout_passes=False),
    )
    return kern(*args)


def _spmm_split(rowp, colp, valp, x, bnd):
    scratch = [
        pltpu.VMEM((PHB, K), I32),
        pltpu.VMEM((PHB, K), I32),
        pltpu.VMEM((PHB, K), F32),
        pltpu.VMEM((K, D), F32),
        pltpu.VMEM((K, D), F32),
        pltpu.VMEM((K,), I32),
        pltpu.VMEM((L,), I32),
        pltpu.VMEM_SHARED((U + L, D), F32),
        pltpu.SemaphoreType.DMA,
        pltpu.SemaphoreType.DMA,
    ]

    def body(row_h, col_h, val_h, x_h, bnd_h, out_h, rowall, colall, valall,
             gbufa, gbufb, rowloc, bndv, acc, sema, semb):
        c = lax.axis_index("c")
        s = lax.axis_index("s")
        pltpu.sync_copy(bnd_h, bndv)
        b1 = bndv[pl.ds(0, L)][0]
        lo = jnp.where(c == 0, 0, b1)
        hi = jnp.where(c == 0, b1, E)
        i0 = lo // K
        ihi = (hi + K - 1) // K
        cpt = (ihi - i0 + NS - 1) // NS
        start = i0 + s * cpt
        _zero_acc(acc, gbufa, s, U + L)
        plsc.subcore_barrier()
        rbase = c * U

        for ph in range(NPHS):
            start_ph = start + ph * NCHP
            start8 = start_ph // 8 * 8
            doff = start_ph - start8

            @pl.when(ph * NCHP < cpt)
            def _():
                pltpu.sync_copy(row_h.at[pl.ds(start8, PHB)], rowall)
                pltpu.sync_copy(col_h.at[pl.ds(start8, PHB)], colall)
                pltpu.sync_copy(val_h.at[pl.ds(start8, PHB)], valall)

                def pred(t):
                    return jnp.logical_and(ph * NCHP + t < cpt,
                                           (start_ph + t) * K < hi)

                def issue(t, buf, sem):
                    @pl.when(pred(t))
                    def _():
                        pltpu.async_copy(x_h.at[colall.at[t + doff]], buf,
                                         sem)

                def proc(t, buf, sem):
                    @pl.when(pred(t))
                    def _():
                        tt = t + doff
                        pltpu.make_async_copy(x_h.at[colall.at[tt]], buf,
                                              sem).wait()
                        _scale_rows(buf, valall, tt)
                        for g in range(K // L):
                            r = rowall[tt, pl.ds(g * L, L)]
                            local = r - rbase
                            inb = jnp.logical_and(local >= 0, local < U)
                            rowloc[pl.ds(g * L, L)] = jnp.where(inb, local, U)
                        pltpu.sync_copy(buf, acc.at[rowloc], add=True)

                issue(0, gbufa, sema)

                def pair(p, carry):
                    t0 = 2 * p
                    issue(t0 + 1, gbufb, semb)
                    proc(t0, gbufa, sema)
                    issue(t0 + 2, gbufa, sema)
                    proc(t0 + 1, gbufb, semb)
                    return carry

                lax.fori_loop(0, (NCHP - 1) // 2, pair, 0)
                proc(NCHP - 1, gbufa, sema)
        plsc.subcore_barrier()
        _spread_copy(lambda o, n: acc.at[pl.ds(o, n)],
                     lambda o, n: out_h.at[pl.ds(c * U + o, n)], s, U)

    kern = pl.kernel(
        body,
        out_type=jax.ShapeDtypeStruct((N2, D), F32),
        mesh=plsc.VectorSubcoreMesh(**_MESH),
        scratch_types=scratch,
        compiler_params=pltpu.CompilerParams(needs_layout_passes=False),
    )
    return kern(rowp, colp, valp, x, bnd)


def _sims_kernel(row3, col3, a_norm):
    scratch = [
        pltpu.VMEM((NCHP, K), I32),
        pltpu.VMEM((NCHP, K), I32),
        pltpu.VMEM((NCHP, K), F32),
        pltpu.VMEM((K, D), F32),
        pltpu.VMEM((K, D), F32),
        pltpu.VMEM((K, D), F32),
        pltpu.VMEM((K, D), F32),
        pltpu.VMEM((K, D), F32),
        pltpu.VMEM((K, D), F32),
        pltpu.VMEM((WIN, D), F32),
        pltpu.VMEM((WIN, D), F32),
        pltpu.VMEM((WIN, D), F32),
        pltpu.VMEM((ACTB,), I32),
        pltpu.VMEM((L,), I32),
        pltpu.VMEM_SHARED((UPAD,), F32),
        pltpu.SemaphoreType.DMA,
        pltpu.SemaphoreType.DMA,
        pltpu.SemaphoreType.DMA,
    ]

    def body(row_h, col_h, an_h, prn_h, diag_h, act_h, cnt_h, rowall, colall,
             prnall, browa, bcola, browb, bcolb, browc, bcolc, wina, winb,
             winc, actb, cntb, dacc, sem1, sem2, sem3):
        c = lax.axis_index("c")
        s = lax.axis_index("s")
        w = s * NC + c
        _zero_vec(prnall.at[0], K)
        dpt = UPAD // NS
        for q in range(dpt // K):
            pltpu.sync_copy(prnall.at[0],
                            dacc.at[pl.ds(s * dpt + q * K, K)])
        plsc.subcore_barrier()
        lanes = lax.iota(I32, L)

        def chunk_meta(j):
            r0 = rowall[j, pl.ds(0, L)][0]
            rlast = rowall[j, pl.ds(K - L, L)][L - 1]
            w0 = jnp.minimum(r0 // 8 * 8, U - WIN)
            return w0, rlast < w0 + WIN

        def issue(j, win, brow, bcol, sem):
            w0, small = chunk_meta(j)

            @pl.when(small)
            def _():
                pltpu.async_copy(an_h.at[pl.ds(w0, WIN)], win, sem)

            @pl.when(jnp.logical_not(small))
            def _():
                pltpu.async_copy(an_h.at[rowall.at[j]], brow, sem)

            pltpu.async_copy(an_h.at[colall.at[j]], bcol, sem)

        def drain(j, win, brow, bcol, sem):
            w0, small = chunk_meta(j)

            @pl.when(small)
            def _():
                pltpu.make_async_copy(an_h.at[pl.ds(w0, WIN)], win,
                                      sem).wait()

            @pl.when(jnp.logical_not(small))
            def _():
                pltpu.make_async_copy(an_h.at[rowall.at[j]], brow,
                                      sem).wait()

            pltpu.make_async_copy(an_h.at[colall.at[j]], bcol, sem).wait()

        def phase(ph, cursor):
            pltpu.sync_copy(row_h.at[w, ph], rowall)
            pltpu.sync_copy(col_h.at[w, ph], colall)
            bufs = ((wina, browa, bcola, sem1), (winb, browb, bcolb, sem2),
                    (winc, browc, bcolc, sem3))
            issue(0, *bufs[0])
            issue(1, *bufs[1])

            def compute(j, cur, win, brow, bcol):
                w0, small = chunk_meta(j)

                def mkgroup(rowref, use_win):
                    def group(g, nsurv):
                        rr = rowall[j, pl.ds(g * L, L)] - w0
                        dots = jnp.zeros((L,), F32)
                        for e in range(L):
                            erow = rr[e] if use_win else g * L + e
                            part = jnp.zeros((L,), F32)
                            for d in range(D // L):
                                part = (part
                                        + rowref[erow, pl.ds(d * L, L)]
                                        * bcol[g * L + e, pl.ds(d * L, L)])
                            dot = jnp.sum(part)
                            dots = jnp.where(lanes == e, dot, dots)
                        simv = (dots + 1.0) * 0.5
                        keep = simv >= 0.8
                        pv = jnp.where(keep, simv, 0.0)
                        prnall[j, pl.ds(g * L, L)] = pv
                        return nsurv + jnp.sum(jnp.where(keep, 1, 0))
                    return group

                nsurv = lax.cond(
                    small,
                    lambda: lax.fori_loop(0, K // L, mkgroup(win, True), 0),
                    lambda: lax.fori_loop(0, K // L, mkgroup(brow, False), 0))
                pltpu.sync_copy(prnall.at[j], dacc.at[rowall.at[j]], add=True)
                jg = ph * NCHP + j
                plsc.store_compressed(actb.at[pl.ds(cur, L)],
                                      jnp.broadcast_to(jg, (L,)),
                                      mask=lanes == 0)
                return cur + jnp.where(nsurv > 0, 1, 0)

            def trip(p, cur):
                for b in range(3):
                    j = 3 * p + b
                    w_, br_, bc_, s_ = bufs[b]
                    drain(j, w_, br_, bc_, s_)

                    @pl.when(j + 2 < NCHP)
                    def _():
                        issue(j + 2, *bufs[(b + 2) % 3])

                    cur = compute(j, cur, w_, br_, bc_)
                return cur

            cur = lax.fori_loop(0, 8, trip, cursor)
            jt = NCHP - 1
            w_, br_, bc_, s_ = bufs[jt % 3]
            drain(jt, w_, br_, bc_, s_)
            cur = compute(jt, cur, w_, br_, bc_)
            pltpu.sync_copy(prnall, prn_h.at[w, ph])
            return cur

        cursor = lax.fori_loop(0, NPH, phase, 0)
        cntb[pl.ds(0, L)] = jnp.where(lanes == 0, cursor, 0)
        pltpu.sync_copy(actb, act_h.at[w])
        pltpu.sync_copy(cntb, cnt_h.at[w])
        plsc.subcore_barrier()
        pltpu.sync_copy(dacc.at[pl.ds(s * dpt, dpt)],
                        diag_h.at[c, pl.ds(s * dpt, dpt)])

    kern = pl.kernel(
        body,
        out_type=[jax.ShapeDtypeStruct((NW, NPH, NCHP, K), F32),
                  jax.ShapeDtypeStruct((NC, UPAD), F32),
                  jax.ShapeDtypeStruct((NW, ACTB), I32),
                  jax.ShapeDtypeStruct((NW, L), I32)],
        mesh=plsc.VectorSubcoreMesh(**_MESH),
        scratch_types=scratch,
        compiler_params=pltpu.CompilerParams(needs_layout_passes=False),
    )
    return kern(row3, col3, a_norm)


def _spmm_compact(rowf, colf, valf, act, cnt, x0, x1, diag0, diag1):
    scratch = [
        pltpu.VMEM((ACTB,), I32),
        pltpu.VMEM((L,), I32),
        pltpu.VMEM((K,), I32),
        pltpu.VMEM((K,), I32),
        pltpu.VMEM((K,), F32),
        pltpu.VMEM((K,), F32),
        pltpu.VMEM((K,), F32),
        pltpu.VMEM((K, D), F32),
        pltpu.VMEM_SHARED((U, D), F32),
        pltpu.SemaphoreType.DMA,
    ]

    def body(row_h, col_h, val_h, act_h, cnt_h, x0_h, x1_h, dg0_h, dg1_h,
             out_h, actb, cntb, rowb, colb, valb, d0b, d1b, gbuf, acc, sem):
        c = lax.axis_index("c")
        s = lax.axis_index("s")
        w = s * NC + c
        lanes = lax.iota(I32, L)
        pltpu.sync_copy(act_h.at[w], actb)
        pltpu.sync_copy(cnt_h.at[w], cntb)
        nact = jnp.sum(jnp.where(lanes == 0, cntb[pl.ds(0, L)], 0))
        _zero_acc(acc, gbuf, s, U)
        plsc.subcore_barrier()

        def agroup(g, carry):
            av = actb[pl.ds(g * L, L)]
            for e in range(L):
                jloc = av[e]

                @pl.when(g * L + e < nact)
                def _():
                    gcid = w * NCHK + jloc
                    pltpu.sync_copy(row_h.at[gcid, 0], rowb)
                    pltpu.sync_copy(col_h.at[gcid, 0], colb)
                    pltpu.sync_copy(val_h.at[gcid, 0], valb)
                    pltpu.async_copy(dg0_h.at[rowb], d0b, sem).wait()
                    pltpu.async_copy(dg1_h.at[rowb], d1b, sem).wait()
                    for gg in range(K // L):
                        sl = pl.ds(gg * L, L)
                        valb[sl] = valb[sl] / (d0b[sl] + d1b[sl] + 1e-7)
                    pltpu.async_copy(x0_h.at[colb], gbuf, sem).wait()
                    pltpu.async_copy(x1_h.at[colb], gbuf, sem,
                                     add=True).wait()

                    def sgroup(gg, c2):
                        vv = valb[pl.ds(gg * L, L)]
                        for ee in range(L):
                            vb = jnp.broadcast_to(vv[ee], (L,))
                            r = gg * L + ee
                            for d in range(D // L):
                                gbuf[r, pl.ds(d * L, L)] = (
                                    gbuf[r, pl.ds(d * L, L)] * vb)
                        return c2

                    lax.fori_loop(0, K // L, sgroup, 0)
                    pltpu.sync_copy(gbuf, acc.at[rowb], add=True)

            return carry

        lax.fori_loop(0, ACTB // L, agroup, 0)
        plsc.subcore_barrier()
        _spread_copy(lambda o, n: acc.at[pl.ds(o, n)],
                     lambda o, n: out_h.at[c, pl.ds(o, n)], s, U)

    kern = pl.kernel(
        body,
        out_type=jax.ShapeDtypeStruct((NC, U, D), F32),
        mesh=plsc.VectorSubcoreMesh(**_MESH),
        scratch_types=scratch,
        compiler_params=pltpu.CompilerParams(needs_layout_passes=False),
    )
    return kern(rowf, colf, valf, act, cnt, x0, x1, diag0, diag1)


_BU = 1000


def _tc_normalize(u2p):
    def body(pref, oref):
        x = pref[0] + pref[1]
        n = jnp.sqrt(jnp.sum(x * x, axis=1, keepdims=True))
        oref[...] = x / jnp.maximum(n, 1e-8)

    return pl.pallas_call(
        body,
        grid=(U // _BU,),
        in_specs=[pl.BlockSpec((NC, _BU, D), lambda i: (0, i, 0))],
        out_specs=pl.BlockSpec((_BU, D), lambda i: (i, 0)),
        out_shape=jax.ShapeDtypeStruct((U, D), F32),
    )(u2p)


def _tc_combine_social(pp, diag2):
    def body(pref, dref, oref):
        dsum = dref[0] + dref[1] + 1e-7
        oref[...] = (pref[0] + pref[1]) / dsum

    return pl.pallas_call(
        body,
        grid=(U // _BU,),
        in_specs=[pl.BlockSpec((NC, _BU, D), lambda i: (0, i, 0)),
                  pl.BlockSpec((NC, _BU, 1), lambda i: (0, i, 0))],
        out_specs=pl.BlockSpec((_BU, D), lambda i: (i, 0)),
        out_shape=jax.ShapeDtypeStruct((U, D), F32),
    )(pp, diag2.reshape(NC, U, 1))


def _tc_sview_ego(p1, p2, p3, user_emb):
    def body(a, b, c, ue, sv, eg):
        m = (a[0] + a[1] + b[0] + b[1] + c[0] + c[1]) / 3.0
        sv[...] = m
        eg[...] = ue[...] + m

    ps = pl.BlockSpec((NC, _BU, D), lambda i: (0, i, 0))
    bs = pl.BlockSpec((_BU, D), lambda i: (i, 0))
    return pl.pallas_call(
        body,
        grid=(U // _BU,),
        in_specs=[ps, ps, ps, bs],
        out_specs=[bs, bs],
        out_shape=[jax.ShapeDtypeStruct((U, D), F32),
                   jax.ShapeDtypeStruct((U, D), F32)],
    )(p1, p2, p3, user_emb)


def _tc_perturb(raw, noise):
    def body(rref, nref, oref):
        nz = nref[...]
        nn = nz / jnp.maximum(
            jnp.sqrt(jnp.sum(nz * nz, axis=1, keepdims=True)), 1e-12)
        r = rref[...]
        oref[...] = r + jnp.sign(r) * nn * 0.1

    bs = pl.BlockSpec((_BU, D), lambda i: (i, 0))
    return pl.pallas_call(
        body,
        grid=(N2 // _BU,),
        in_specs=[bs, bs],
        out_specs=bs,
        out_shape=jax.ShapeDtypeStruct((N2, D), F32),
    )(raw, noise)


def _tc_final(e0u, e1u, e2u, e0i, e1i, e2i, sview, w1t, w2t):
    def body(a, b, c, ai, bi, ci, sv, w1, w2, ua, iv):
        uv1 = (a[...] + b[...] + c[...]) / 3.0
        iv[...] = (ai[...] + bi[...] + ci[...]) / 3.0
        svv = sv[...]
        z = (jnp.dot(uv1, w1[...], preferred_element_type=F32)
             + jnp.dot(svv, w2[...], preferred_element_type=F32))
        gu = jax.nn.sigmoid(z)
        ua[...] = gu * svv + (1.0 - gu) * uv1

    bs = pl.BlockSpec((_BU, D), lambda i: (i, 0))
    ws = pl.BlockSpec((D, D), lambda i: (0, 0))
    return pl.pallas_call(
        body,
        grid=(U // _BU,),
        in_specs=[bs, bs, bs, bs, bs, bs, bs, ws, ws],
        out_specs=[bs, bs],
        out_shape=[jax.ShapeDtypeStruct((U, D), F32),
                   jax.ShapeDtypeStruct((U, D), F32)],
    )(e0u, e1u, e2u, e0i, e1i, e2i, sview, w1t, w2t)


def kernel(user_emb, item_emb, social_row, social_col, social_val,
           adj_row, adj_col, adj_val, W1, W2):
    del social_val
    srow = social_row.astype(I32)
    scol = social_col.astype(I32)
    arow = adj_row.astype(I32)
    acol = adj_col.astype(I32)
    aval = adj_val.astype(F32)

    srow3 = srow.reshape(NW, NPH, NCHP, K)
    scol3 = scol.reshape(NW, NPH, NCHP, K)

    u2p = _spmm_partial(srow3, scol3, user_emb)
    a_norm = _tc_normalize(u2p)

    pruned3, diag_pad, act, cnt = _sims_kernel(srow3, scol3, a_norm)
    diag0 = diag_pad[0]
    diag1 = diag_pad[1]
    srowf = srow.reshape(TOTCH, 1, K)
    scolf = scol.reshape(TOTCH, 1, K)
    valf = pruned3.reshape(TOTCH, 1, K)

    x0, x1 = user_emb, jnp.zeros((U, D), F32)
    pps = []
    for _ in range(3):
        pp = _spmm_compact(srowf, scolf, valf, act, cnt, x0, x1, diag0, diag1)
        pps.append(pp)
        x0, x1 = pp[0], pp[1]
    user_sview, ego_user = _tc_sview_ego(pps[0], pps[1], pps[2], user_emb)
    ego0 = jnp.concatenate([ego_user, item_emb], axis=0)

    pad_rows = NPAD - NCHT
    arowp = jnp.concatenate(
        [arow.reshape(NCHT, K), jnp.zeros((pad_rows, K), I32)], axis=0)
    acolp = jnp.concatenate(
        [acol.reshape(NCHT, K), jnp.zeros((pad_rows, K), I32)], axis=0)
    avalp = jnp.concatenate(
        [aval.reshape(NCHT, K), jnp.zeros((pad_rows, K), F32)], axis=0)
    b1 = jnp.searchsorted(arow, U).astype(I32)
    bnd = jnp.broadcast_to(b1, (L,)).astype(I32)

    nkey = jax.random.key(42)
    egos = [ego0]
    ego = ego0
    for k in range(2):
        raw = _spmm_split(arowp, acolp, avalp, ego, bnd)
        noise = jax.random.uniform(jax.random.fold_in(nkey, k), (N2, D),
                                   dtype=F32)
        ego = _tc_perturb(raw, noise)
        egos.append(ego)

    user_all, item_v1 = _tc_final(
        egos[0][:U], egos[1][:U], egos[2][:U],
        egos[0][U:], egos[1][U:], egos[2][U:],
        user_sview, W1.T, W2.T)
    return (user_all, item_v1, jnp.stack(egos, axis=1))


# speedup vs baseline: 1.0700x; 1.0267x over previous
import functools

import jax
import jax.numpy as jnp
from jax import lax
from jax.experimental import pallas as pl
from jax.experimental.pallas import tpu as pltpu
from jax.experimental.pallas import tpu_sc as plsc

U = 10000
NI = 10000
D = 128
E = 320000
N2 = U + NI
NC = 2
NS = 16
NW = NC * NS
L = 16
EPW = E // NW
K = 80
NCHK = EPW // K
NPH = 5
NCHP = NCHK // NPH
NPHS = 10
PHB = 32
NCHT = E // K
CAPC = 250
CAPB = 264
NPAD = 4280
UPAD = 10240
TOTCH = E // K
ACTB = 144
WIN = 24
F32 = jnp.float32
I32 = jnp.int32

_MESH = dict(core_axis_name="c", subcore_axis_name="s", num_cores=NC,
             num_subcores=NS)


def _zero_vec(ref, n):
    z = jnp.zeros((L,), ref.dtype)

    def body(i, carry):
        ref[pl.ds(i * L, L)] = z
        return carry

    lax.fori_loop(0, n // L, body, 0)


def _zero_rows(ref, rows):
    z = jnp.zeros((L,), F32)

    def body(i, carry):
        for d in range(D // L):
            ref[i, pl.ds(d * L, L)] = z
        return carry

    lax.fori_loop(0, rows, body, 0)


def _zero_acc_slice(acc, gbuf, base, total):
    _zero_rows(gbuf, K)
    off = 0
    while off < total:
        step = min(K, total - off)
        pltpu.sync_copy(gbuf.at[pl.ds(0, step)],
                        acc.at[pl.ds(base + off, step)])
        off += step


def _scale_rows(gbuf, vals, j):
    def body(g, carry):
        vv = vals[j, pl.ds(g * L, L)]
        for e in range(L):
            vb = jnp.broadcast_to(vv[e], (L,))
            row = g * L + e
            for d in range(D // L):
                gbuf[row, pl.ds(d * L, L)] = gbuf[row, pl.ds(d * L, L)] * vb
        return carry

    lax.fori_loop(0, K // L, body, 0)


def _spread_copy(src_fn, dst_fn, s, total):
    bq = (total // NS) // 8 * 8
    rem = total - NS * bq
    base = s * bq
    pltpu.sync_copy(src_fn(base, bq), dst_fn(base, bq))
    if rem:
        @pl.when(s == NS - 1)
        def _():
            pltpu.sync_copy(src_fn(NS * bq, rem), dst_fn(NS * bq, rem))


def _zero_acc(acc, gbuf, s, total):
    _zero_rows(gbuf, K)
    bq = (total // NS) // 8 * 8
    rem = total - NS * bq
    base = s * bq
    off = 0
    while off < bq:
        step = min(K, bq - off)
        pltpu.sync_copy(gbuf.at[pl.ds(0, step)],
                        acc.at[pl.ds(base + off, step)])
        off += step
    if rem:
        @pl.when(s == NS - 1)
        def _():
            pltpu.sync_copy(gbuf.at[pl.ds(0, rem)],
                            acc.at[pl.ds(NS * bq, rem)])


def _spmm_partial(row3, col3, x, val3=None):
    scale = val3 is not None

    scratch = [
        pltpu.VMEM((NCHP, K), I32),
        pltpu.VMEM((NCHP, K), I32),
        pltpu.VMEM((NCHP, K), F32),
        pltpu.VMEM((K, D), F32),
        pltpu.VMEM((K, D), F32),
        pltpu.VMEM_SHARED((U, D), F32),
        pltpu.SemaphoreType.DMA,
        pltpu.SemaphoreType.DMA,
    ]

    def body(row_h, col_h, val_h, x_h, out_h, rowall, colall, valall, gbufa,
             gbufb, acc, sema, semb):
        c = lax.axis_index("c")
        s = lax.axis_index("s")
        w = s * NC + c
        _zero_acc(acc, gbufa, s, U)
        plsc.subcore_barrier()
        bufs = ((gbufa, sema), (gbufb, semb))

        def proc(j, buf):
            if scale:
                _scale_rows(buf, valall, j)
            pltpu.sync_copy(buf, acc.at[rowall.at[j]], add=True)

        for ph in range(NPH):
            pltpu.sync_copy(row_h.at[w, ph], rowall)
            pltpu.sync_copy(col_h.at[w, ph], colall)
            if scale:
                pltpu.sync_copy(val_h.at[w, ph], valall)
            pltpu.async_copy(x_h.at[colall.at[0]], gbufa, sema)

            def pair(p, carry):
                j0 = 2 * p
                pltpu.make_async_copy(x_h.at[colall.at[j0]], gbufa,
                                      sema).wait()
                pltpu.async_copy(x_h.at[colall.at[j0 + 1]], gbufb, semb)
                proc(j0, gbufa)
                pltpu.make_async_copy(x_h.at[colall.at[j0 + 1]], gbufb,
                                      semb).wait()
                pltpu.async_copy(x_h.at[colall.at[j0 + 2]], gbufa, sema)
                proc(j0 + 1, gbufb)
                return carry

            lax.fori_loop(0, (NCHP - 1) // 2, pair, 0)
            pltpu.make_async_copy(x_h.at[colall.at[NCHP - 1]], gbufa,
                                  sema).wait()
            proc(NCHP - 1, gbufa)
        plsc.subcore_barrier()
        _spread_copy(lambda o, n: acc.at[pl.ds(o, n)],
                     lambda o, n: out_h.at[c, pl.ds(o, n)], s, U)

    args = [row3, col3, val3 if scale else col3, x]
    kern = pl.kernel(
        body,
        out_type=jax.ShapeDtypeStruct((NC, U, D), F32),
        mesh=plsc.VectorSubcoreMesh(**_MESH),
        scratch_types=scratch,
        compiler_params=pltpu.CompilerParams(needs_layout_passes=False),
    )
    return kern(*args)


def _spmm_split(rowp, colp, valp, x, bnd):
    scratch = [
        pltpu.VMEM((PHB, K), I32),
        pltpu.VMEM((PHB, K), I32),
        pltpu.VMEM((PHB, K), F32),
        pltpu.VMEM((K, D), F32),
        pltpu.VMEM((K, D), F32),
        pltpu.VMEM((K,), I32),
        pltpu.VMEM((L,), I32),
        pltpu.VMEM_SHARED((U + L, D), F32),
        pltpu.SemaphoreType.DMA,
        pltpu.SemaphoreType.DMA,
    ]

    def body(row_h, col_h, val_h, x_h, bnd_h, out_h, rowall, colall, valall,
             gbufa, gbufb, rowloc, bndv, acc, sema, semb):
        c = lax.axis_index("c")
        s = lax.axis_index("s")
        pltpu.sync_copy(bnd_h, bndv)
        b1 = bndv[pl.ds(0, L)][0]
        lo = jnp.where(c == 0, 0, b1)
        hi = jnp.where(c == 0, b1, E)
        i0 = lo // K
        ihi = (hi + K - 1) // K
        cpt = (ihi - i0 + NS - 1) // NS
        start = i0 + s * cpt
        _zero_acc(acc, gbufa, s, U + L)
        plsc.subcore_barrier()
        rbase = c * U

        for ph in range(NPHS):
            start_ph = start + ph * NCHP
            start8 = start_ph // 8 * 8
            doff = start_ph - start8

            @pl.when(ph * NCHP < cpt)
            def _():
                pltpu.sync_copy(row_h.at[pl.ds(start8, PHB)], rowall)
                pltpu.sync_copy(col_h.at[pl.ds(start8, PHB)], colall)
                pltpu.sync_copy(val_h.at[pl.ds(start8, PHB)], valall)

                def pred(t):
                    return jnp.logical_and(ph * NCHP + t < cpt,
                                           (start_ph + t) * K < hi)

                def issue(t, buf, sem):
                    @pl.when(pred(t))
                    def _():
                        pltpu.async_copy(x_h.at[colall.at[t + doff]], buf,
                                         sem)

                def proc(t, buf, sem):
                    @pl.when(pred(t))
                    def _():
                        tt = t + doff
                        pltpu.make_async_copy(x_h.at[colall.at[tt]], buf,
                                              sem).wait()
                        _scale_rows(buf, valall, tt)
                        for g in range(K // L):
                            r = rowall[tt, pl.ds(g * L, L)]
                            local = r - rbase
                            inb = jnp.logical_and(local >= 0, local < U)
                            rowloc[pl.ds(g * L, L)] = jnp.where(inb, local, U)
                        pltpu.sync_copy(buf, acc.at[rowloc], add=True)

                issue(0, gbufa, sema)

                def pair(p, carry):
                    t0 = 2 * p
                    issue(t0 + 1, gbufb, semb)
                    proc(t0, gbufa, sema)
                    issue(t0 + 2, gbufa, sema)
                    proc(t0 + 1, gbufb, semb)
                    return carry

                lax.fori_loop(0, (NCHP - 1) // 2, pair, 0)
                proc(NCHP - 1, gbufa, sema)
        plsc.subcore_barrier()
        _spread_copy(lambda o, n: acc.at[pl.ds(o, n)],
                     lambda o, n: out_h.at[pl.ds(c * U + o, n)], s, U)

    kern = pl.kernel(
        body,
        out_type=jax.ShapeDtypeStruct((N2, D), F32),
        mesh=plsc.VectorSubcoreMesh(**_MESH),
        scratch_types=scratch,
        compiler_params=pltpu.CompilerParams(needs_layout_passes=False),
    )
    return kern(rowp, colp, valp, x, bnd)


def _sims_kernel(row3, col3, a_norm):
    scratch = [
        pltpu.VMEM((NCHP, K), I32),
        pltpu.VMEM((NCHP, K), I32),
        pltpu.VMEM((NCHP, K), F32),
        pltpu.VMEM((K, D), F32),
        pltpu.VMEM((K, D), F32),
        pltpu.VMEM((K, D), F32),
        pltpu.VMEM((K, D), F32),
        pltpu.VMEM((WIN, D), F32),
        pltpu.VMEM((WIN, D), F32),
        pltpu.VMEM((ACTB,), I32),
        pltpu.VMEM((L,), I32),
        pltpu.VMEM_SHARED((UPAD,), F32),
        pltpu.SemaphoreType.DMA,
        pltpu.SemaphoreType.DMA,
    ]

    def body(row_h, col_h, an_h, prn_h, diag_h, act_h, cnt_h, rowall, colall,
             prnall, browa, bcola, browb, bcolb, wina, winb, actb, cntb, dacc,
             sem1, sem2):
        c = lax.axis_index("c")
        s = lax.axis_index("s")
        w = s * NC + c
        _zero_vec(prnall.at[0], K)
        dpt = UPAD // NS
        for q in range(dpt // K):
            pltpu.sync_copy(prnall.at[0],
                            dacc.at[pl.ds(s * dpt + q * K, K)])
        plsc.subcore_barrier()
        lanes = lax.iota(I32, L)

        def chunk_meta(j):
            r0 = rowall[j, pl.ds(0, L)][0]
            rlast = rowall[j, pl.ds(K - L, L)][L - 1]
            w0 = jnp.minimum(r0 // 8 * 8, U - WIN)
            return w0, rlast < w0 + WIN

        def issue(j, win, brow, bcol, sem):
            w0, small = chunk_meta(j)

            @pl.when(small)
            def _():
                pltpu.async_copy(an_h.at[pl.ds(w0, WIN)], win, sem)

            @pl.when(jnp.logical_not(small))
            def _():
                pltpu.async_copy(an_h.at[rowall.at[j]], brow, sem)

            pltpu.async_copy(an_h.at[colall.at[j]], bcol, sem)

        def drain(j, win, brow, bcol, sem):
            w0, small = chunk_meta(j)

            @pl.when(small)
            def _():
                pltpu.make_async_copy(an_h.at[pl.ds(w0, WIN)], win,
                                      sem).wait()

            @pl.when(jnp.logical_not(small))
            def _():
                pltpu.make_async_copy(an_h.at[rowall.at[j]], brow,
                                      sem).wait()

            pltpu.make_async_copy(an_h.at[colall.at[j]], bcol, sem).wait()

        def phase(ph, cursor):
            pltpu.sync_copy(row_h.at[w, ph], rowall)
            pltpu.sync_copy(col_h.at[w, ph], colall)
            issue(0, wina, browa, bcola, sem1)

            def compute(j, cur, win, brow, bcol):
                w0, small = chunk_meta(j)

                def mkgroup(rowref, use_win):
                    def group(g, nsurv):
                        rr = rowall[j, pl.ds(g * L, L)] - w0
                        dots = jnp.zeros((L,), F32)
                        for e in range(L):
                            erow = rr[e] if use_win else g * L + e
                            part = jnp.zeros((L,), F32)
                            for d in range(D // L):
                                part = (part
                                        + rowref[erow, pl.ds(d * L, L)]
                                        * bcol[g * L + e, pl.ds(d * L, L)])
                            dot = jnp.sum(part)
                            dots = jnp.where(lanes == e, dot, dots)
                        simv = (dots + 1.0) * 0.5
                        keep = simv >= 0.8
                        pv = jnp.where(keep, simv, 0.0)
                        prnall[j, pl.ds(g * L, L)] = pv
                        return nsurv + jnp.sum(jnp.where(keep, 1, 0))
                    return group

                nsurv = lax.cond(
                    small,
                    lambda: lax.fori_loop(0, K // L, mkgroup(win, True), 0),
                    lambda: lax.fori_loop(0, K // L, mkgroup(brow, False), 0))
                pltpu.sync_copy(prnall.at[j], dacc.at[rowall.at[j]], add=True)
                jg = ph * NCHP + j
                plsc.store_compressed(actb.at[pl.ds(cur, L)],
                                      jnp.broadcast_to(jg, (L,)),
                                      mask=lanes == 0)
                return cur + jnp.where(nsurv > 0, 1, 0)

            def pair(p, cur):
                j0 = 2 * p
                drain(j0, wina, browa, bcola, sem1)
                issue(j0 + 1, winb, browb, bcolb, sem2)
                cur = compute(j0, cur, wina, browa, bcola)
                drain(j0 + 1, winb, browb, bcolb, sem2)
                issue(j0 + 2, wina, browa, bcola, sem1)
                cur = compute(j0 + 1, cur, winb, browb, bcolb)
                return cur

            cur = lax.fori_loop(0, (NCHP - 1) // 2, pair, cursor)
            drain(NCHP - 1, wina, browa, bcola, sem1)
            cur = compute(NCHP - 1, cur, wina, browa, bcola)
            pltpu.sync_copy(prnall, prn_h.at[w, ph])
            return cur

        cursor = lax.fori_loop(0, NPH, phase, 0)
        cntb[pl.ds(0, L)] = jnp.where(lanes == 0, cursor, 0)
        pltpu.sync_copy(actb, act_h.at[w])
        pltpu.sync_copy(cntb, cnt_h.at[w])
        plsc.subcore_barrier()
        pltpu.sync_copy(dacc.at[pl.ds(s * dpt, dpt)],
                        diag_h.at[c, pl.ds(s * dpt, dpt)])

    kern = pl.kernel(
        body,
        out_type=[jax.ShapeDtypeStruct((NW, NPH, NCHP, K), F32),
                  jax.ShapeDtypeStruct((NC, UPAD), F32),
                  jax.ShapeDtypeStruct((NW, ACTB), I32),
                  jax.ShapeDtypeStruct((NW, L), I32)],
        mesh=plsc.VectorSubcoreMesh(**_MESH),
        scratch_types=scratch,
        compiler_params=pltpu.CompilerParams(needs_layout_passes=False),
    )
    return kern(row3, col3, a_norm)


def _spmm_compact(rowf, colf, valf, act, cnt, x0, x1, diag0, diag1):
    scratch = [
        pltpu.VMEM((ACTB,), I32),
        pltpu.VMEM((L,), I32),
        pltpu.VMEM((K,), I32),
        pltpu.VMEM((K,), I32),
        pltpu.VMEM((K,), F32),
        pltpu.VMEM((K,), F32),
        pltpu.VMEM((K,), F32),
        pltpu.VMEM((K, D), F32),
        pltpu.VMEM_SHARED((U, D), F32),
        pltpu.SemaphoreType.DMA,
    ]

    def body(row_h, col_h, val_h, act_h, cnt_h, x0_h, x1_h, dg0_h, dg1_h,
             out_h, actb, cntb, rowb, colb, valb, d0b, d1b, gbuf, acc, sem):
        c = lax.axis_index("c")
        s = lax.axis_index("s")
        w = s * NC + c
        lanes = lax.iota(I32, L)
        pltpu.sync_copy(act_h.at[w], actb)
        pltpu.sync_copy(cnt_h.at[w], cntb)
        nact = jnp.sum(jnp.where(lanes == 0, cntb[pl.ds(0, L)], 0))
        _zero_acc(acc, gbuf, s, U)
        plsc.subcore_barrier()

        def agroup(g, carry):
            av = actb[pl.ds(g * L, L)]
            for e in range(L):
                jloc = av[e]

                @pl.when(g * L + e < nact)
                def _():
                    gcid = w * NCHK + jloc
                    pltpu.sync_copy(row_h.at[gcid, 0], rowb)
                    pltpu.sync_copy(col_h.at[gcid, 0], colb)
                    pltpu.sync_copy(val_h.at[gcid, 0], valb)
                    pltpu.async_copy(dg0_h.at[rowb], d0b, sem).wait()
                    pltpu.async_copy(dg1_h.at[rowb], d1b, sem).wait()
                    for gg in range(K // L):
                        sl = pl.ds(gg * L, L)
                        valb[sl] = valb[sl] / (d0b[sl] + d1b[sl] + 1e-7)
                    pltpu.async_copy(x0_h.at[colb], gbuf, sem).wait()
                    pltpu.async_copy(x1_h.at[colb], gbuf, sem,
                                     add=True).wait()

                    def sgroup(gg, c2):
                        vv = valb[pl.ds(gg * L, L)]
                        for ee in range(L):
                            vb = jnp.broadcast_to(vv[ee], (L,))
                            r = gg * L + ee
                            for d in range(D // L):
                                gbuf[r, pl.ds(d * L, L)] = (
                                    gbuf[r, pl.ds(d * L, L)] * vb)
                        return c2

                    lax.fori_loop(0, K // L, sgroup, 0)
                    pltpu.sync_copy(gbuf, acc.at[rowb], add=True)

            return carry

        lax.fori_loop(0, ACTB // L, agroup, 0)
        plsc.subcore_barrier()
        _spread_copy(lambda o, n: acc.at[pl.ds(o, n)],
                     lambda o, n: out_h.at[c, pl.ds(o, n)], s, U)

    kern = pl.kernel(
        body,
        out_type=jax.ShapeDtypeStruct((NC, U, D), F32),
        mesh=plsc.VectorSubcoreMesh(**_MESH),
        scratch_types=scratch,
        compiler_params=pltpu.CompilerParams(needs_layout_passes=False),
    )
    return kern(rowf, colf, valf, act, cnt, x0, x1, diag0, diag1)


_BU = 1000


def _tc_normalize(u2p):
    def body(pref, oref):
        x = pref[0] + pref[1]
        n = jnp.sqrt(jnp.sum(x * x, axis=1, keepdims=True))
        oref[...] = x / jnp.maximum(n, 1e-8)

    return pl.pallas_call(
        body,
        grid=(U // _BU,),
        in_specs=[pl.BlockSpec((NC, _BU, D), lambda i: (0, i, 0))],
        out_specs=pl.BlockSpec((_BU, D), lambda i: (i, 0)),
        out_shape=jax.ShapeDtypeStruct((U, D), F32),
    )(u2p)


def _tc_combine_social(pp, diag2):
    def body(pref, dref, oref):
        dsum = dref[0] + dref[1] + 1e-7
        oref[...] = (pref[0] + pref[1]) / dsum

    return pl.pallas_call(
        body,
        grid=(U // _BU,),
        in_specs=[pl.BlockSpec((NC, _BU, D), lambda i: (0, i, 0)),
                  pl.BlockSpec((NC, _BU, 1), lambda i: (0, i, 0))],
        out_specs=pl.BlockSpec((_BU, D), lambda i: (i, 0)),
        out_shape=jax.ShapeDtypeStruct((U, D), F32),
    )(pp, diag2.reshape(NC, U, 1))


def _tc_sview_ego(p1, p2, p3, user_emb):
    def body(a, b, c, ue, sv, eg):
        m = (a[0] + a[1] + b[0] + b[1] + c[0] + c[1]) / 3.0
        sv[...] = m
        eg[...] = ue[...] + m

    ps = pl.BlockSpec((NC, _BU, D), lambda i: (0, i, 0))
    bs = pl.BlockSpec((_BU, D), lambda i: (i, 0))
    return pl.pallas_call(
        body,
        grid=(U // _BU,),
        in_specs=[ps, ps, ps, bs],
        out_specs=[bs, bs],
        out_shape=[jax.ShapeDtypeStruct((U, D), F32),
                   jax.ShapeDtypeStruct((U, D), F32)],
    )(p1, p2, p3, user_emb)


def _tc_perturb(raw, noise):
    def body(rref, nref, oref):
        nz = nref[...]
        nn = nz / jnp.maximum(
            jnp.sqrt(jnp.sum(nz * nz, axis=1, keepdims=True)), 1e-12)
        r = rref[...]
        oref[...] = r + jnp.sign(r) * nn * 0.1

    bs = pl.BlockSpec((_BU, D), lambda i: (i, 0))
    return pl.pallas_call(
        body,
        grid=(N2 // _BU,),
        in_specs=[bs, bs],
        out_specs=bs,
        out_shape=jax.ShapeDtypeStruct((N2, D), F32),
    )(raw, noise)


def _tc_final(e0u, e1u, e2u, e0i, e1i, e2i, sview, w1t, w2t):
    def body(a, b, c, ai, bi, ci, sv, w1, w2, ua, iv):
        uv1 = (a[...] + b[...] + c[...]) / 3.0
        iv[...] = (ai[...] + bi[...] + ci[...]) / 3.0
        svv = sv[...]
        z = (jnp.dot(uv1, w1[...], preferred_element_type=F32)
             + jnp.dot(svv, w2[...], preferred_element_type=F32))
        gu = jax.nn.sigmoid(z)
        ua[...] = gu * svv + (1.0 - gu) * uv1

    bs = pl.BlockSpec((_BU, D), lambda i: (i, 0))
    ws = pl.BlockSpec((D, D), lambda i: (0, 0))
    return pl.pallas_call(
        body,
        grid=(U // _BU,),
        in_specs=[bs, bs, bs, bs, bs, bs, bs, ws, ws],
        out_specs=[bs, bs],
        out_shape=[jax.ShapeDtypeStruct((U, D), F32),
                   jax.ShapeDtypeStruct((U, D), F32)],
    )(e0u, e1u, e2u, e0i, e1i, e2i, sview, w1t, w2t)


def kernel(user_emb, item_emb, social_row, social_col, social_val,
           adj_row, adj_col, adj_val, W1, W2):
    del social_val
    srow = social_row.astype(I32)
    scol = social_col.astype(I32)
    arow = adj_row.astype(I32)
    acol = adj_col.astype(I32)
    aval = adj_val.astype(F32)

    srow3 = srow.reshape(NW, NPH, NCHP, K)
    scol3 = scol.reshape(NW, NPH, NCHP, K)

    u2p = _spmm_partial(srow3, scol3, user_emb)
    a_norm = _tc_normalize(u2p)

    pruned3, diag_pad, act, cnt = _sims_kernel(srow3, scol3, a_norm)
    diag0 = diag_pad[0]
    diag1 = diag_pad[1]
    srowf = srow.reshape(TOTCH, 1, K)
    scolf = scol.reshape(TOTCH, 1, K)
    valf = pruned3.reshape(TOTCH, 1, K)

    x0, x1 = user_emb, jnp.zeros((U, D), F32)
    pps = []
    for _ in range(3):
        pp = _spmm_compact(srowf, scolf, valf, act, cnt, x0, x1, diag0, diag1)
        pps.append(pp)
        x0, x1 = pp[0], pp[1]
    user_sview, ego_user = _tc_sview_ego(pps[0], pps[1], pps[2], user_emb)
    ego0 = jnp.concatenate([ego_user, item_emb], axis=0)

    pad_rows = NPAD - NCHT
    arowp = jnp.concatenate(
        [arow.reshape(NCHT, K), jnp.zeros((pad_rows, K), I32)], axis=0)
    acolp = jnp.concatenate(
        [acol.reshape(NCHT, K), jnp.zeros((pad_rows, K), I32)], axis=0)
    avalp = jnp.concatenate(
        [aval.reshape(NCHT, K), jnp.zeros((pad_rows, K), F32)], axis=0)
    b1 = jnp.searchsorted(arow, U).astype(I32)
    bnd = jnp.broadcast_to(b1, (L,)).astype(I32)

    nkey = jax.random.key(42)
    egos = [ego0]
    ego = ego0
    for k in range(2):
        raw = _spmm_split(arowp, acolp, avalp, ego, bnd)
        noise = jax.random.uniform(jax.random.fold_in(nkey, k), (N2, D),
                                   dtype=F32)
        ego = _tc_perturb(raw, noise)
        egos.append(ego)

    user_all, item_v1 = _tc_final(
        egos[0][:U], egos[1][:U], egos[2][:U],
        egos[0][U:], egos[1][U:], egos[2][U:],
        user_sview, W1.T, W2.T)
    return (user_all, item_v1, jnp.stack(egos, axis=1))
